# baseline (speedup 1.0000x reference)
import jax
import jax.numpy as jnp
from jax.experimental import pallas as pl
from jax.experimental.pallas import tpu as pltpu

N = 10000
E = 320000
H = 128
L = 20
N_CLASS = 10
N_LAYER = 3


def _lin(p, x):
    return x @ p["W"] + p["b"]


def _bn(x, g, b):
    m = jnp.mean(x, axis=0)
    v = jnp.var(x, axis=0)
    return (x - m) / jnp.sqrt(v + 1e-5) * g + b


def _lstm(x, p):
    n = x.shape[0]
    xT = jnp.swapaxes(x, 0, 1)
    h0 = jnp.zeros((n, H), x.dtype)
    c0 = jnp.zeros((n, H), x.dtype)

    def step(carry, xt):
        hh, cc = carry
        gates = xt @ p["W_ih"].T + hh @ p["W_hh"].T + p["b"]
        i_, f_, g_, o_ = jnp.split(gates, 4, axis=-1)
        i_ = jax.nn.sigmoid(i_)
        f_ = jax.nn.sigmoid(f_)
        g_ = jnp.tanh(g_)
        o_ = jax.nn.sigmoid(o_)
        cc = f_ * cc + i_ * g_
        hh = o_ * jnp.tanh(cc)
        return (hh, cc), hh

    _, hs = jax.lax.scan(step, (h0, c0), xT)
    return jnp.swapaxes(hs, 0, 1)


def _mlp_head_kernel(y_ref, w0_ref, b0_ref, w1_ref, b1_ref, w2_ref, b2_ref,
                     out_ref):
    y = y_ref[...]
    y = jnp.maximum(y @ w0_ref[...] + b0_ref[...], 0.0)
    y = jnp.maximum(y @ w1_ref[...] + b1_ref[...], 0.0)
    out_ref[...] = y @ w2_ref[...] + b2_ref[...]


def _mlp_head(y, mlp):
    n = y.shape[0]
    blk = 2000
    grid = (n // blk,)
    w0, b0 = mlp[0]["W"], mlp[0]["b"]
    w1, b1 = mlp[1]["W"], mlp[1]["b"]
    w2, b2 = mlp[2]["W"], mlp[2]["b"]
    w2p = jnp.zeros((w2.shape[0], 128), w2.dtype).at[:, :N_CLASS].set(w2)
    b2p = jnp.zeros((128,), b2.dtype).at[:N_CLASS].set(b2)
    out = pl.pallas_call(
        _mlp_head_kernel,
        grid=grid,
        in_specs=[
            pl.BlockSpec((blk, H), lambda i: (i, 0)),
            pl.BlockSpec((H, 64), lambda i: (0, 0)),
            pl.BlockSpec((64,), lambda i: (0,)),
            pl.BlockSpec((64, 32), lambda i: (0, 0)),
            pl.BlockSpec((32,), lambda i: (0,)),
            pl.BlockSpec((32, 128), lambda i: (0, 0)),
            pl.BlockSpec((128,), lambda i: (0,)),
        ],
        out_specs=pl.BlockSpec((blk, 128), lambda i: (i, 0)),
        out_shape=jax.ShapeDtypeStruct((n, 128), y.dtype),
    )(y, w0, b0, w1, b1, w2p, b2p)
    return out[:, :N_CLASS]


def kernel(h, e, text, snorm_n, snorm_e, edge_index, text_length,
           graph_node_size, graph_edge_size, params):
    txt = jnp.take(params["text_emb"], text, axis=0)
    hs_f = _lstm(txt, params["lstm_f"])
    idx = (text_length - 1)[:, None, None]
    hf = jnp.take_along_axis(hs_f, idx, axis=1)[:, 0, :]
    pos = text_length[:, None] - 1 - jnp.arange(L)[None, :]
    mask = (pos >= 0).astype(txt.dtype)
    rev = jnp.take_along_axis(txt, jnp.clip(pos, 0, L - 1)[:, :, None], axis=1) * mask[:, :, None]
    hs_b = _lstm(rev, params["lstm_b"])
    hb = jnp.take_along_axis(hs_b, idx, axis=1)[:, 0, :]
    text_emb = 0.5 * (hf + hb)
    text_emb = text_emb / jnp.maximum(jnp.linalg.norm(text_emb, axis=1, keepdims=True), 1e-12)
    hcur = _lin(params["h_emb"], h) + text_emb
    ecur = _lin(params["e_emb"], e)
    src = edge_index[0]
    dst = edge_index[1]
    n_nodes = h.shape[0]
    all_h = [hcur]
    for li in range(N_LAYER):
        lay = params["layers"][li]
        Ah = _lin(lay["A"], hcur)
        Bh = _lin(lay["B"], hcur)
        Ce = _lin(lay["C"], ecur)
        Dh = _lin(lay["D"], hcur)
        Eh = _lin(lay["E"], hcur)
        e_new = Dh[src] + Eh[dst] + Ce
        sigma = jax.nn.sigmoid(e_new)
        num = jax.ops.segment_sum(sigma * Bh[src], dst, num_segments=n_nodes)
        den = jax.ops.segment_sum(sigma, dst, num_segments=n_nodes)
        hn = Ah + num / (den + 1e-6)
        hn = _bn(hn * snorm_n, lay["bn_h_g"], lay["bn_h_b"])
        hn = hcur + jax.nn.relu(hn)
        en = _bn(e_new * snorm_e, lay["bn_e_g"], lay["bn_e_b"])
        en = ecur + jax.nn.relu(en)
        all_h.append(hn)
        hcur = jax.nn.relu(_lin(params["dense"][li], jnp.concatenate(all_h, axis=1)))
        ecur = en
    return _mlp_head(hcur, params["mlp"])


# speedup vs baseline: 1.1829x; 1.1829x over previous
import functools

import jax
import jax.numpy as jnp
from jax import lax
from jax.experimental import pallas as pl
from jax.experimental.pallas import tpu as pltpu
from jax.experimental.pallas import tpu_sc as plsc

N = 10000
E = 320000
H = 128
L = 20
N_CLASS = 10
N_LAYER = 3


_TIL = 16
_EPT = E // _TIL
_CH = 512
_NCH = _EPT // _CH
_TAIL = _EPT - _NCH * _CH
_NHALF = 5120
_DUMP = _NHALF
_ROWS_T = _NHALF // _TIL


def _remap(buf, n, base_n):
    for v in range(n // 16):
        d = buf[pl.ds(v * 16, 16)] - base_n
        ok = (d >= 0) & (d < _NHALF)
        buf[pl.ds(v * 16, 16)] = jnp.where(ok, d, _DUMP)


def _segsum2_kernel(msg_hbm, sig_hbm, idx_hbm, z_hbm, num_hbm, den_hbm,
                    vals_v, i0, i1, i2, i3, it, accum):
    c = lax.axis_index("c")
    s = lax.axis_index("s")
    base_n = c * _NHALF
    idx_bufs = (i0, i1, i2, i3)

    def zero_accum():
        pltpu.sync_copy(z_hbm, accum.at[pl.ds(s * _ROWS_T, _ROWS_T)])

    def run(vhbm):
        base = s * _EPT

        def chunk(k, carry):
            cb = base + k * _CH
            pltpu.sync_copy(vhbm.at[pl.ds(cb, _CH)], vals_v)
            for j in range(4):
                pltpu.sync_copy(idx_hbm.at[pl.ds(cb + j * 128, 128)],
                                idx_bufs[j])
            for j in range(4):
                _remap(idx_bufs[j], 128, base_n)
                pltpu.sync_copy(vals_v.at[pl.ds(j * 128, 128)],
                                accum.at[idx_bufs[j]], add=True)
            return carry

        lax.fori_loop(0, _NCH, chunk, 0)
        tb = base + _NCH * _CH
        pltpu.sync_copy(vhbm.at[pl.ds(tb, _TAIL)], vals_v.at[pl.ds(0, _TAIL)])
        pltpu.sync_copy(idx_hbm.at[pl.ds(tb, _TAIL)], it)
        _remap(it, _TAIL, base_n)
        pltpu.sync_copy(vals_v.at[pl.ds(0, _TAIL)], accum.at[it], add=True)

    def copy_out(dst_hbm):
        rb = s * _ROWS_T
        pltpu.sync_copy(accum.at[pl.ds(rb, _ROWS_T)],
                        vals_v.at[pl.ds(0, _ROWS_T)])
        pltpu.sync_copy(vals_v.at[pl.ds(0, _ROWS_T)],
                        dst_hbm.at[pl.ds(base_n + rb, _ROWS_T)])

    zero_accum()
    plsc.subcore_barrier()
    run(msg_hbm)
    plsc.subcore_barrier()
    copy_out(num_hbm)
    plsc.subcore_barrier()
    zero_accum()
    plsc.subcore_barrier()
    run(sig_hbm)
    plsc.subcore_barrier()
    copy_out(den_hbm)


@jax.jit
def _segsum2(msg, sig, dst):
    mesh = plsc.VectorSubcoreMesh(core_axis_name="c", subcore_axis_name="s")
    zeros = jnp.zeros((_ROWS_T, H), jnp.float32)
    k = functools.partial(
        pl.kernel,
        mesh=mesh,
        out_type=(jax.ShapeDtypeStruct((2 * _NHALF, H), jnp.float32),
                  jax.ShapeDtypeStruct((2 * _NHALF, H), jnp.float32)),
        scratch_types=[
            pltpu.VMEM((_CH, H), jnp.float32),
            pltpu.VMEM((128,), jnp.int32),
            pltpu.VMEM((128,), jnp.int32),
            pltpu.VMEM((128,), jnp.int32),
            pltpu.VMEM((128,), jnp.int32),
            pltpu.VMEM((_TAIL,), jnp.int32),
            pltpu.VMEM_SHARED((_NHALF + 8, H), jnp.float32),
        ],
    )(_segsum2_kernel)
    num, den = k(msg, sig, dst, zeros)
    return num[:N], den[:N]


def _lin(p, x):
    return x @ p["W"] + p["b"]


def _bn(x, g, b):
    m = jnp.mean(x, axis=0)
    v = jnp.var(x, axis=0)
    return (x - m) / jnp.sqrt(v + 1e-5) * g + b


def _lstm(x, p):
    n = x.shape[0]
    xT = jnp.swapaxes(x, 0, 1)
    h0 = jnp.zeros((n, H), x.dtype)
    c0 = jnp.zeros((n, H), x.dtype)

    def step(carry, xt):
        hh, cc = carry
        gates = xt @ p["W_ih"].T + hh @ p["W_hh"].T + p["b"]
        i_, f_, g_, o_ = jnp.split(gates, 4, axis=-1)
        i_ = jax.nn.sigmoid(i_)
        f_ = jax.nn.sigmoid(f_)
        g_ = jnp.tanh(g_)
        o_ = jax.nn.sigmoid(o_)
        cc = f_ * cc + i_ * g_
        hh = o_ * jnp.tanh(cc)
        return (hh, cc), hh

    _, hs = jax.lax.scan(step, (h0, c0), xT)
    return jnp.swapaxes(hs, 0, 1)


def _mlp_head_kernel(y_ref, w0_ref, b0_ref, w1_ref, b1_ref, w2_ref, b2_ref,
                     out_ref):
    y = y_ref[...]
    y = jnp.maximum(y @ w0_ref[...] + b0_ref[...], 0.0)
    y = jnp.maximum(y @ w1_ref[...] + b1_ref[...], 0.0)
    out_ref[...] = y @ w2_ref[...] + b2_ref[...]


def _mlp_head(y, mlp):
    n = y.shape[0]
    blk = 2000
    grid = (n // blk,)
    w0, b0 = mlp[0]["W"], mlp[0]["b"]
    w1, b1 = mlp[1]["W"], mlp[1]["b"]
    w2, b2 = mlp[2]["W"], mlp[2]["b"]
    w2p = jnp.zeros((w2.shape[0], 128), w2.dtype).at[:, :N_CLASS].set(w2)
    b2p = jnp.zeros((128,), b2.dtype).at[:N_CLASS].set(b2)
    out = pl.pallas_call(
        _mlp_head_kernel,
        grid=grid,
        in_specs=[
            pl.BlockSpec((blk, H), lambda i: (i, 0)),
            pl.BlockSpec((H, 64), lambda i: (0, 0)),
            pl.BlockSpec((64,), lambda i: (0,)),
            pl.BlockSpec((64, 32), lambda i: (0, 0)),
            pl.BlockSpec((32,), lambda i: (0,)),
            pl.BlockSpec((32, 128), lambda i: (0, 0)),
            pl.BlockSpec((128,), lambda i: (0,)),
        ],
        out_specs=pl.BlockSpec((blk, 128), lambda i: (i, 0)),
        out_shape=jax.ShapeDtypeStruct((n, 128), y.dtype),
    )(y, w0, b0, w1, b1, w2p, b2p)
    return out[:, :N_CLASS]


def kernel(h, e, text, snorm_n, snorm_e, edge_index, text_length,
           graph_node_size, graph_edge_size, params):
    txt = jnp.take(params["text_emb"], text, axis=0)
    hs_f = _lstm(txt, params["lstm_f"])
    idx = (text_length - 1)[:, None, None]
    hf = jnp.take_along_axis(hs_f, idx, axis=1)[:, 0, :]
    pos = text_length[:, None] - 1 - jnp.arange(L)[None, :]
    mask = (pos >= 0).astype(txt.dtype)
    rev = jnp.take_along_axis(txt, jnp.clip(pos, 0, L - 1)[:, :, None], axis=1) * mask[:, :, None]
    hs_b = _lstm(rev, params["lstm_b"])
    hb = jnp.take_along_axis(hs_b, idx, axis=1)[:, 0, :]
    text_emb = 0.5 * (hf + hb)
    text_emb = text_emb / jnp.maximum(jnp.linalg.norm(text_emb, axis=1, keepdims=True), 1e-12)
    hcur = _lin(params["h_emb"], h) + text_emb
    ecur = _lin(params["e_emb"], e)
    src = edge_index[0]
    dst = edge_index[1]
    n_nodes = h.shape[0]
    all_h = [hcur]
    for li in range(N_LAYER):
        lay = params["layers"][li]
        Ah = _lin(lay["A"], hcur)
        Bh = _lin(lay["B"], hcur)
        Ce = _lin(lay["C"], ecur)
        Dh = _lin(lay["D"], hcur)
        Eh = _lin(lay["E"], hcur)
        e_new = Dh[src] + Eh[dst] + Ce
        sigma = jax.nn.sigmoid(e_new)
        num, den = _segsum2(sigma * Bh[src], sigma, dst)
        hn = Ah + num / (den + 1e-6)
        hn = _bn(hn * snorm_n, lay["bn_h_g"], lay["bn_h_b"])
        hn = hcur + jax.nn.relu(hn)
        en = _bn(e_new * snorm_e, lay["bn_e_g"], lay["bn_e_b"])
        en = ecur + jax.nn.relu(en)
        all_h.append(hn)
        hcur = jax.nn.relu(_lin(params["dense"][li], jnp.concatenate(all_h, axis=1)))
        ecur = en
    return _mlp_head(hcur, params["mlp"])


# speedup vs baseline: 1.7764x; 1.5017x over previous
import functools

import jax
import jax.numpy as jnp
from jax import lax
from jax.experimental import pallas as pl
from jax.experimental.pallas import tpu as pltpu
from jax.experimental.pallas import tpu_sc as plsc

N = 10000
E = 320000
H = 128
L = 20
N_CLASS = 10
N_LAYER = 3


_TIL = 16
_EPT = E // _TIL
_CH = 512
_NCH = _EPT // _CH
_TAIL = _EPT - _NCH * _CH
_NHALF = 5120
_DUMP = _NHALF
_ROWS_T = _NHALF // _TIL


def _remap(buf, n, base_n):
    for v in range(n // 16):
        d = buf[pl.ds(v * 16, 16)] - base_n
        ok = (d >= 0) & (d < _NHALF)
        buf[pl.ds(v * 16, 16)] = jnp.where(ok, d, _DUMP)


def _segsum2_kernel(msg_hbm, sig_hbm, idx_hbm, z_hbm, num_hbm, den_hbm,
                    vals_v, i0, i1, i2, i3, it, accum):
    c = lax.axis_index("c")
    s = lax.axis_index("s")
    base_n = c * _NHALF
    idx_bufs = (i0, i1, i2, i3)

    def zero_accum():
        pltpu.sync_copy(z_hbm, accum.at[pl.ds(s * _ROWS_T, _ROWS_T)])

    def run(vhbm):
        base = s * _EPT

        def chunk(k, carry):
            cb = base + k * _CH
            pltpu.sync_copy(vhbm.at[pl.ds(cb, _CH)], vals_v)
            for j in range(4):
                pltpu.sync_copy(idx_hbm.at[pl.ds(cb + j * 128, 128)],
                                idx_bufs[j])
            for j in range(4):
                _remap(idx_bufs[j], 128, base_n)
                pltpu.sync_copy(vals_v.at[pl.ds(j * 128, 128)],
                                accum.at[idx_bufs[j]], add=True)
            return carry

        lax.fori_loop(0, _NCH, chunk, 0)
        tb = base + _NCH * _CH
        pltpu.sync_copy(vhbm.at[pl.ds(tb, _TAIL)], vals_v.at[pl.ds(0, _TAIL)])
        pltpu.sync_copy(idx_hbm.at[pl.ds(tb, _TAIL)], it)
        _remap(it, _TAIL, base_n)
        pltpu.sync_copy(vals_v.at[pl.ds(0, _TAIL)], accum.at[it], add=True)

    def copy_out(dst_hbm):
        rb = s * _ROWS_T
        pltpu.sync_copy(accum.at[pl.ds(rb, _ROWS_T)],
                        vals_v.at[pl.ds(0, _ROWS_T)])
        pltpu.sync_copy(vals_v.at[pl.ds(0, _ROWS_T)],
                        dst_hbm.at[pl.ds(base_n + rb, _ROWS_T)])

    zero_accum()
    plsc.subcore_barrier()
    run(msg_hbm)
    plsc.subcore_barrier()
    copy_out(num_hbm)
    plsc.subcore_barrier()
    zero_accum()
    plsc.subcore_barrier()
    run(sig_hbm)
    plsc.subcore_barrier()
    copy_out(den_hbm)


@jax.jit
def _segsum2(msg, sig, dst):
    mesh = plsc.VectorSubcoreMesh(core_axis_name="c", subcore_axis_name="s")
    zeros = jnp.zeros((_ROWS_T, H), jnp.float32)
    k = functools.partial(
        pl.kernel,
        mesh=mesh,
        out_type=(jax.ShapeDtypeStruct((2 * _NHALF, H), jnp.float32),
                  jax.ShapeDtypeStruct((2 * _NHALF, H), jnp.float32)),
        scratch_types=[
            pltpu.VMEM((_CH, H), jnp.float32),
            pltpu.VMEM((128,), jnp.int32),
            pltpu.VMEM((128,), jnp.int32),
            pltpu.VMEM((128,), jnp.int32),
            pltpu.VMEM((128,), jnp.int32),
            pltpu.VMEM((_TAIL,), jnp.int32),
            pltpu.VMEM_SHARED((_NHALF + 8, H), jnp.float32),
        ],
    )(_segsum2_kernel)
    num, den = k(msg, sig, dst, zeros)
    return num[:N], den[:N]


_GW = 32
_GEPW = E // _GW
_GCH = 256
_GNCH = _GEPW // _GCH
_GTAIL = _GEPW - _GNCH * _GCH


def _gather3_kernel(dh_hbm, eh_hbm, bh_hbm, src_hbm, dst_hbm,
                    ds_hbm, es_hbm, bs_hbm,
                    dbuf, ebuf, bbuf, s0, s1, d0, d1, st, dt):
    c = lax.axis_index("c")
    s = lax.axis_index("s")
    w = s * 2 + c
    base = w * _GEPW
    sbufs = (s0, s1)
    dbufs = (d0, d1)

    def chunk(k, carry):
        cb = base + k * _GCH
        for j in range(2):
            pltpu.sync_copy(src_hbm.at[pl.ds(cb + j * 128, 128)], sbufs[j])
            pltpu.sync_copy(dst_hbm.at[pl.ds(cb + j * 128, 128)], dbufs[j])
        for j in range(2):
            pltpu.sync_copy(dh_hbm.at[sbufs[j]], dbuf.at[pl.ds(j * 128, 128)])
            pltpu.sync_copy(eh_hbm.at[dbufs[j]], ebuf.at[pl.ds(j * 128, 128)])
            pltpu.sync_copy(bh_hbm.at[sbufs[j]], bbuf.at[pl.ds(j * 128, 128)])
        pltpu.sync_copy(dbuf, ds_hbm.at[pl.ds(cb, _GCH)])
        pltpu.sync_copy(ebuf, es_hbm.at[pl.ds(cb, _GCH)])
        pltpu.sync_copy(bbuf, bs_hbm.at[pl.ds(cb, _GCH)])
        return carry

    lax.fori_loop(0, _GNCH, chunk, 0)
    tb = base + _GNCH * _GCH
    pltpu.sync_copy(src_hbm.at[pl.ds(tb, _GTAIL)], st)
    pltpu.sync_copy(dst_hbm.at[pl.ds(tb, _GTAIL)], dt)
    pltpu.sync_copy(dh_hbm.at[st], dbuf.at[pl.ds(0, _GTAIL)])
    pltpu.sync_copy(eh_hbm.at[dt], ebuf.at[pl.ds(0, _GTAIL)])
    pltpu.sync_copy(bh_hbm.at[st], bbuf.at[pl.ds(0, _GTAIL)])
    pltpu.sync_copy(dbuf.at[pl.ds(0, _GTAIL)], ds_hbm.at[pl.ds(tb, _GTAIL)])
    pltpu.sync_copy(ebuf.at[pl.ds(0, _GTAIL)], es_hbm.at[pl.ds(tb, _GTAIL)])
    pltpu.sync_copy(bbuf.at[pl.ds(0, _GTAIL)], bs_hbm.at[pl.ds(tb, _GTAIL)])


@jax.jit
def _gather3(dh, eh, bh, src, dst):
    mesh = plsc.VectorSubcoreMesh(core_axis_name="c", subcore_axis_name="s")
    sh = jax.ShapeDtypeStruct((E, H), jnp.float32)
    k = functools.partial(
        pl.kernel,
        mesh=mesh,
        out_type=(sh, sh, sh),
        scratch_types=[
            pltpu.VMEM((_GCH, H), jnp.float32),
            pltpu.VMEM((_GCH, H), jnp.float32),
            pltpu.VMEM((_GCH, H), jnp.float32),
            pltpu.VMEM((128,), jnp.int32),
            pltpu.VMEM((128,), jnp.int32),
            pltpu.VMEM((128,), jnp.int32),
            pltpu.VMEM((128,), jnp.int32),
            pltpu.VMEM((_GTAIL,), jnp.int32),
            pltpu.VMEM((_GTAIL,), jnp.int32),
        ],
    )(_gather3_kernel)
    return k(dh, eh, bh, src, dst)


def _lin(p, x):
    return x @ p["W"] + p["b"]


def _bn(x, g, b):
    m = jnp.mean(x, axis=0)
    v = jnp.var(x, axis=0)
    return (x - m) / jnp.sqrt(v + 1e-5) * g + b


def _lstm(x, p):
    n = x.shape[0]
    xT = jnp.swapaxes(x, 0, 1)
    h0 = jnp.zeros((n, H), x.dtype)
    c0 = jnp.zeros((n, H), x.dtype)

    def step(carry, xt):
        hh, cc = carry
        gates = xt @ p["W_ih"].T + hh @ p["W_hh"].T + p["b"]
        i_, f_, g_, o_ = jnp.split(gates, 4, axis=-1)
        i_ = jax.nn.sigmoid(i_)
        f_ = jax.nn.sigmoid(f_)
        g_ = jnp.tanh(g_)
        o_ = jax.nn.sigmoid(o_)
        cc = f_ * cc + i_ * g_
        hh = o_ * jnp.tanh(cc)
        return (hh, cc), hh

    _, hs = jax.lax.scan(step, (h0, c0), xT)
    return jnp.swapaxes(hs, 0, 1)


def _mlp_head_kernel(y_ref, w0_ref, b0_ref, w1_ref, b1_ref, w2_ref, b2_ref,
                     out_ref):
    y = y_ref[...]
    y = jnp.maximum(y @ w0_ref[...] + b0_ref[...], 0.0)
    y = jnp.maximum(y @ w1_ref[...] + b1_ref[...], 0.0)
    out_ref[...] = y @ w2_ref[...] + b2_ref[...]


def _mlp_head(y, mlp):
    n = y.shape[0]
    blk = 2000
    grid = (n // blk,)
    w0, b0 = mlp[0]["W"], mlp[0]["b"]
    w1, b1 = mlp[1]["W"], mlp[1]["b"]
    w2, b2 = mlp[2]["W"], mlp[2]["b"]
    w2p = jnp.zeros((w2.shape[0], 128), w2.dtype).at[:, :N_CLASS].set(w2)
    b2p = jnp.zeros((128,), b2.dtype).at[:N_CLASS].set(b2)
    out = pl.pallas_call(
        _mlp_head_kernel,
        grid=grid,
        in_specs=[
            pl.BlockSpec((blk, H), lambda i: (i, 0)),
            pl.BlockSpec((H, 64), lambda i: (0, 0)),
            pl.BlockSpec((64,), lambda i: (0,)),
            pl.BlockSpec((64, 32), lambda i: (0, 0)),
            pl.BlockSpec((32,), lambda i: (0,)),
            pl.BlockSpec((32, 128), lambda i: (0, 0)),
            pl.BlockSpec((128,), lambda i: (0,)),
        ],
        out_specs=pl.BlockSpec((blk, 128), lambda i: (i, 0)),
        out_shape=jax.ShapeDtypeStruct((n, 128), y.dtype),
    )(y, w0, b0, w1, b1, w2p, b2p)
    return out[:, :N_CLASS]


def kernel(h, e, text, snorm_n, snorm_e, edge_index, text_length,
           graph_node_size, graph_edge_size, params):
    txt = jnp.take(params["text_emb"], text, axis=0)
    hs_f = _lstm(txt, params["lstm_f"])
    idx = (text_length - 1)[:, None, None]
    hf = jnp.take_along_axis(hs_f, idx, axis=1)[:, 0, :]
    pos = text_length[:, None] - 1 - jnp.arange(L)[None, :]
    mask = (pos >= 0).astype(txt.dtype)
    rev = jnp.take_along_axis(txt, jnp.clip(pos, 0, L - 1)[:, :, None], axis=1) * mask[:, :, None]
    hs_b = _lstm(rev, params["lstm_b"])
    hb = jnp.take_along_axis(hs_b, idx, axis=1)[:, 0, :]
    text_emb = 0.5 * (hf + hb)
    text_emb = text_emb / jnp.maximum(jnp.linalg.norm(text_emb, axis=1, keepdims=True), 1e-12)
    hcur = _lin(params["h_emb"], h) + text_emb
    ecur = _lin(params["e_emb"], e)
    src = edge_index[0]
    dst = edge_index[1]
    n_nodes = h.shape[0]
    all_h = [hcur]
    for li in range(N_LAYER):
        lay = params["layers"][li]
        Ah = _lin(lay["A"], hcur)
        Bh = _lin(lay["B"], hcur)
        Ce = _lin(lay["C"], ecur)
        Dh = _lin(lay["D"], hcur)
        Eh = _lin(lay["E"], hcur)
        DS, ES, BS = _gather3(Dh, Eh, Bh, src, dst)
        e_new = DS + ES + Ce
        sigma = jax.nn.sigmoid(e_new)
        num, den = _segsum2(sigma * BS, sigma, dst)
        hn = Ah + num / (den + 1e-6)
        hn = _bn(hn * snorm_n, lay["bn_h_g"], lay["bn_h_b"])
        hn = hcur + jax.nn.relu(hn)
        en = _bn(e_new * snorm_e, lay["bn_e_g"], lay["bn_e_b"])
        en = ecur + jax.nn.relu(en)
        all_h.append(hn)
        hcur = jax.nn.relu(_lin(params["dense"][li], jnp.concatenate(all_h, axis=1)))
        ecur = en
    return _mlp_head(hcur, params["mlp"])


# speedup vs baseline: 2.3069x; 1.2987x over previous
import functools

import jax
import jax.numpy as jnp
from jax import lax
from jax.experimental import pallas as pl
from jax.experimental.pallas import tpu as pltpu
from jax.experimental.pallas import tpu_sc as plsc

N = 10000
E = 320000
H = 128
L = 20
N_CLASS = 10
N_LAYER = 3


_TIL = 16
_EPT = E // _TIL
_CH = 512
_NCH = _EPT // _CH
_TAIL = _EPT - _NCH * _CH
_NHALF = 5120
_DUMP = _NHALF
_ROWS_T = _NHALF // _TIL


def _remap(buf, n, base_n):
    for v in range(n // 16):
        d = buf[pl.ds(v * 16, 16)] - base_n
        ok = (d >= 0) & (d < _NHALF)
        buf[pl.ds(v * 16, 16)] = jnp.where(ok, d, _DUMP)


def _segsum2_kernel(msg_hbm, sig_hbm, idx_hbm, z_hbm, num_hbm, den_hbm,
                    vals_v, i0, i1, i2, i3, it, accum):
    c = lax.axis_index("c")
    s = lax.axis_index("s")
    base_n = c * _NHALF
    idx_bufs = (i0, i1, i2, i3)

    def zero_accum():
        pltpu.sync_copy(z_hbm, accum.at[pl.ds(s * _ROWS_T, _ROWS_T)])

    def run(vhbm):
        base = s * _EPT

        def chunk(k, carry):
            cb = base + k * _CH
            pltpu.sync_copy(vhbm.at[pl.ds(cb, _CH)], vals_v)
            for j in range(4):
                pltpu.sync_copy(idx_hbm.at[pl.ds(cb + j * 128, 128)],
                                idx_bufs[j])
            for j in range(4):
                _remap(idx_bufs[j], 128, base_n)
                pltpu.sync_copy(vals_v.at[pl.ds(j * 128, 128)],
                                accum.at[idx_bufs[j]], add=True)
            return carry

        lax.fori_loop(0, _NCH, chunk, 0)
        tb = base + _NCH * _CH
        pltpu.sync_copy(vhbm.at[pl.ds(tb, _TAIL)], vals_v.at[pl.ds(0, _TAIL)])
        pltpu.sync_copy(idx_hbm.at[pl.ds(tb, _TAIL)], it)
        _remap(it, _TAIL, base_n)
        pltpu.sync_copy(vals_v.at[pl.ds(0, _TAIL)], accum.at[it], add=True)

    def copy_out(dst_hbm):
        rb = s * _ROWS_T
        pltpu.sync_copy(accum.at[pl.ds(rb, _ROWS_T)],
                        vals_v.at[pl.ds(0, _ROWS_T)])
        pltpu.sync_copy(vals_v.at[pl.ds(0, _ROWS_T)],
                        dst_hbm.at[pl.ds(base_n + rb, _ROWS_T)])

    zero_accum()
    plsc.subcore_barrier()
    run(msg_hbm)
    plsc.subcore_barrier()
    copy_out(num_hbm)
    plsc.subcore_barrier()
    zero_accum()
    plsc.subcore_barrier()
    run(sig_hbm)
    plsc.subcore_barrier()
    copy_out(den_hbm)


@jax.jit
def _segsum2(msg, sig, dst):
    mesh = plsc.VectorSubcoreMesh(core_axis_name="c", subcore_axis_name="s")
    zeros = jnp.zeros((_ROWS_T, H), jnp.float32)
    k = functools.partial(
        pl.kernel,
        mesh=mesh,
        out_type=(jax.ShapeDtypeStruct((2 * _NHALF, H), jnp.float32),
                  jax.ShapeDtypeStruct((2 * _NHALF, H), jnp.float32)),
        scratch_types=[
            pltpu.VMEM((_CH, H), jnp.float32),
            pltpu.VMEM((128,), jnp.int32),
            pltpu.VMEM((128,), jnp.int32),
            pltpu.VMEM((128,), jnp.int32),
            pltpu.VMEM((128,), jnp.int32),
            pltpu.VMEM((_TAIL,), jnp.int32),
            pltpu.VMEM_SHARED((_NHALF + 8, H), jnp.float32),
        ],
    )(_segsum2_kernel)
    num, den = k(msg, sig, dst, zeros)
    return num[:N], den[:N]


_GW = 32
_GEPW = E // _GW
_GCH = 256
_GNCH = _GEPW // _GCH
_GTAIL = _GEPW - _GNCH * _GCH


def _gather3_kernel(dh_hbm, eh_hbm, bh_hbm, src_hbm, dst_hbm,
                    ds_hbm, es_hbm, bs_hbm,
                    dbuf, ebuf, bbuf, s0, s1, d0, d1, st, dt):
    c = lax.axis_index("c")
    s = lax.axis_index("s")
    w = s * 2 + c
    base = w * _GEPW
    sbufs = (s0, s1)
    dbufs = (d0, d1)

    def chunk(k, carry):
        cb = base + k * _GCH
        for j in range(2):
            pltpu.sync_copy(src_hbm.at[pl.ds(cb + j * 128, 128)], sbufs[j])
            pltpu.sync_copy(dst_hbm.at[pl.ds(cb + j * 128, 128)], dbufs[j])
        for j in range(2):
            pltpu.sync_copy(dh_hbm.at[sbufs[j]], dbuf.at[pl.ds(j * 128, 128)])
            pltpu.sync_copy(eh_hbm.at[dbufs[j]], ebuf.at[pl.ds(j * 128, 128)])
            pltpu.sync_copy(bh_hbm.at[sbufs[j]], bbuf.at[pl.ds(j * 128, 128)])
        pltpu.sync_copy(dbuf, ds_hbm.at[pl.ds(cb, _GCH)])
        pltpu.sync_copy(ebuf, es_hbm.at[pl.ds(cb, _GCH)])
        pltpu.sync_copy(bbuf, bs_hbm.at[pl.ds(cb, _GCH)])
        return carry

    lax.fori_loop(0, _GNCH, chunk, 0)
    tb = base + _GNCH * _GCH
    pltpu.sync_copy(src_hbm.at[pl.ds(tb, _GTAIL)], st)
    pltpu.sync_copy(dst_hbm.at[pl.ds(tb, _GTAIL)], dt)
    pltpu.sync_copy(dh_hbm.at[st], dbuf.at[pl.ds(0, _GTAIL)])
    pltpu.sync_copy(eh_hbm.at[dt], ebuf.at[pl.ds(0, _GTAIL)])
    pltpu.sync_copy(bh_hbm.at[st], bbuf.at[pl.ds(0, _GTAIL)])
    pltpu.sync_copy(dbuf.at[pl.ds(0, _GTAIL)], ds_hbm.at[pl.ds(tb, _GTAIL)])
    pltpu.sync_copy(ebuf.at[pl.ds(0, _GTAIL)], es_hbm.at[pl.ds(tb, _GTAIL)])
    pltpu.sync_copy(bbuf.at[pl.ds(0, _GTAIL)], bs_hbm.at[pl.ds(tb, _GTAIL)])


@jax.jit
def _gather3(dh, eh, bh, src, dst):
    mesh = plsc.VectorSubcoreMesh(core_axis_name="c", subcore_axis_name="s")
    sh = jax.ShapeDtypeStruct((E, H), jnp.float32)
    k = functools.partial(
        pl.kernel,
        mesh=mesh,
        out_type=(sh, sh, sh),
        scratch_types=[
            pltpu.VMEM((_GCH, H), jnp.float32),
            pltpu.VMEM((_GCH, H), jnp.float32),
            pltpu.VMEM((_GCH, H), jnp.float32),
            pltpu.VMEM((128,), jnp.int32),
            pltpu.VMEM((128,), jnp.int32),
            pltpu.VMEM((128,), jnp.int32),
            pltpu.VMEM((128,), jnp.int32),
            pltpu.VMEM((_GTAIL,), jnp.int32),
            pltpu.VMEM((_GTAIL,), jnp.int32),
        ],
    )(_gather3_kernel)
    return k(dh, eh, bh, src, dst)


_LB = 512
_NPAD = 10240


def _bilstm_kernel(x_ref, h_ref, len_ref,
                   wif_ref, whf_ref, bf_ref, wib_ref, whb_ref, bb_ref,
                   wh_ref, bh_ref, out_ref):
    x = x_ref[...]
    ln = len_ref[...]
    wif, whf, bf = wif_ref[...], whf_ref[...], bf_ref[...]
    wib, whb, bb = wib_ref[...], whb_ref[...], bb_ref[...]

    def step(xt, h, c, wi, wh, b):
        g = jnp.dot(xt, wi, preferred_element_type=jnp.float32)
        g = g + jnp.dot(h, wh, preferred_element_type=jnp.float32) + b
        i_ = jax.nn.sigmoid(g[:, 0:128])
        f_ = jax.nn.sigmoid(g[:, 128:256])
        g_ = jnp.tanh(g[:, 256:384])
        o_ = jax.nn.sigmoid(g[:, 384:512])
        c2 = f_ * c + i_ * g_
        h2 = o_ * jnp.tanh(c2)
        return h2, c2

    z = jnp.zeros((_LB, H), jnp.float32)
    h, c, hf = z, z, z
    for t in range(L):
        xt = x[:, t * H:(t + 1) * H]
        h, c = step(xt, h, c, wif, whf, bf)
        sel = (ln == t + 1).astype(jnp.float32)
        hf = hf + sel * h
    h, c = z, z
    for u in range(L - 1, -1, -1):
        xt = x[:, u * H:(u + 1) * H]
        h2, c2 = step(xt, h, c, wib, whb, bb)
        act = ln > u
        h = jnp.where(act, h2, h)
        c = jnp.where(act, c2, c)
    te = 0.5 * (hf + h)
    nrm = jnp.sqrt(jnp.sum(te * te, axis=1, keepdims=True))
    te = te / jnp.maximum(nrm, 1e-12)
    out_ref[...] = (
        jnp.dot(h_ref[...], wh_ref[...], preferred_element_type=jnp.float32)
        + bh_ref[...] + te)


@jax.jit
def _bilstm_hcur(txt2d, hpad, len2d, pf, pb, ph):
    grid = (_NPAD // _LB,)
    k = pl.pallas_call(
        _bilstm_kernel,
        grid=grid,
        in_specs=[
            pl.BlockSpec((_LB, L * H), lambda i: (i, 0)),
            pl.BlockSpec((_LB, H), lambda i: (i, 0)),
            pl.BlockSpec((_LB, 1), lambda i: (i, 0)),
            pl.BlockSpec((H, 4 * H), lambda i: (0, 0)),
            pl.BlockSpec((H, 4 * H), lambda i: (0, 0)),
            pl.BlockSpec((1, 4 * H), lambda i: (0, 0)),
            pl.BlockSpec((H, 4 * H), lambda i: (0, 0)),
            pl.BlockSpec((H, 4 * H), lambda i: (0, 0)),
            pl.BlockSpec((1, 4 * H), lambda i: (0, 0)),
            pl.BlockSpec((H, H), lambda i: (0, 0)),
            pl.BlockSpec((1, H), lambda i: (0, 0)),
        ],
        out_specs=pl.BlockSpec((_LB, H), lambda i: (i, 0)),
        out_shape=jax.ShapeDtypeStruct((_NPAD, H), jnp.float32),
    )
    return k(txt2d, hpad, len2d,
             pf["W_ih"].T, pf["W_hh"].T, pf["b"][None, :],
             pb["W_ih"].T, pb["W_hh"].T, pb["b"][None, :],
             ph["W"], ph["b"][None, :])


def _lin(p, x):
    return x @ p["W"] + p["b"]


def _bn(x, g, b):
    m = jnp.mean(x, axis=0)
    v = jnp.var(x, axis=0)
    return (x - m) / jnp.sqrt(v + 1e-5) * g + b


def _lstm(x, p):
    n = x.shape[0]
    xT = jnp.swapaxes(x, 0, 1)
    h0 = jnp.zeros((n, H), x.dtype)
    c0 = jnp.zeros((n, H), x.dtype)

    def step(carry, xt):
        hh, cc = carry
        gates = xt @ p["W_ih"].T + hh @ p["W_hh"].T + p["b"]
        i_, f_, g_, o_ = jnp.split(gates, 4, axis=-1)
        i_ = jax.nn.sigmoid(i_)
        f_ = jax.nn.sigmoid(f_)
        g_ = jnp.tanh(g_)
        o_ = jax.nn.sigmoid(o_)
        cc = f_ * cc + i_ * g_
        hh = o_ * jnp.tanh(cc)
        return (hh, cc), hh

    _, hs = jax.lax.scan(step, (h0, c0), xT)
    return jnp.swapaxes(hs, 0, 1)


def _mlp_head_kernel(y_ref, w0_ref, b0_ref, w1_ref, b1_ref, w2_ref, b2_ref,
                     out_ref):
    y = y_ref[...]
    y = jnp.maximum(y @ w0_ref[...] + b0_ref[...], 0.0)
    y = jnp.maximum(y @ w1_ref[...] + b1_ref[...], 0.0)
    out_ref[...] = y @ w2_ref[...] + b2_ref[...]


def _mlp_head(y, mlp):
    n = y.shape[0]
    blk = 2000
    grid = (n // blk,)
    w0, b0 = mlp[0]["W"], mlp[0]["b"]
    w1, b1 = mlp[1]["W"], mlp[1]["b"]
    w2, b2 = mlp[2]["W"], mlp[2]["b"]
    w2p = jnp.zeros((w2.shape[0], 128), w2.dtype).at[:, :N_CLASS].set(w2)
    b2p = jnp.zeros((128,), b2.dtype).at[:N_CLASS].set(b2)
    out = pl.pallas_call(
        _mlp_head_kernel,
        grid=grid,
        in_specs=[
            pl.BlockSpec((blk, H), lambda i: (i, 0)),
            pl.BlockSpec((H, 64), lambda i: (0, 0)),
            pl.BlockSpec((64,), lambda i: (0,)),
            pl.BlockSpec((64, 32), lambda i: (0, 0)),
            pl.BlockSpec((32,), lambda i: (0,)),
            pl.BlockSpec((32, 128), lambda i: (0, 0)),
            pl.BlockSpec((128,), lambda i: (0,)),
        ],
        out_specs=pl.BlockSpec((blk, 128), lambda i: (i, 0)),
        out_shape=jax.ShapeDtypeStruct((n, 128), y.dtype),
    )(y, w0, b0, w1, b1, w2p, b2p)
    return out[:, :N_CLASS]


def kernel(h, e, text, snorm_n, snorm_e, edge_index, text_length,
           graph_node_size, graph_edge_size, params):
    txt = jnp.take(params["text_emb"], text, axis=0)
    txt2d = jnp.zeros((_NPAD, L * H), jnp.float32).at[:N].set(
        txt.reshape(N, L * H))
    hpad = jnp.zeros((_NPAD, H), jnp.float32).at[:N].set(h)
    len2d = jnp.ones((_NPAD, 1), jnp.int32).at[:N, 0].set(text_length.astype(jnp.int32))
    hcur = _bilstm_hcur(txt2d, hpad, len2d,
                        params["lstm_f"], params["lstm_b"],
                        params["h_emb"])[:N]
    ecur = _lin(params["e_emb"], e)
    src = edge_index[0]
    dst = edge_index[1]
    n_nodes = h.shape[0]
    all_h = [hcur]
    for li in range(N_LAYER):
        lay = params["layers"][li]
        Ah = _lin(lay["A"], hcur)
        Bh = _lin(lay["B"], hcur)
        Ce = _lin(lay["C"], ecur)
        Dh = _lin(lay["D"], hcur)
        Eh = _lin(lay["E"], hcur)
        DS, ES, BS = _gather3(Dh, Eh, Bh, src, dst)
        e_new = DS + ES + Ce
        sigma = jax.nn.sigmoid(e_new)
        num, den = _segsum2(sigma * BS, sigma, dst)
        hn = Ah + num / (den + 1e-6)
        hn = _bn(hn * snorm_n, lay["bn_h_g"], lay["bn_h_b"])
        hn = hcur + jax.nn.relu(hn)
        en = _bn(e_new * snorm_e, lay["bn_e_g"], lay["bn_e_b"])
        en = ecur + jax.nn.relu(en)
        all_h.append(hn)
        hcur = jax.nn.relu(_lin(params["dense"][li], jnp.concatenate(all_h, axis=1)))
        ecur = en
    return _mlp_head(hcur, params["mlp"])


# speedup vs baseline: 2.3834x; 1.0332x over previous
import functools

import jax
import jax.numpy as jnp
from jax import lax
from jax.experimental import pallas as pl
from jax.experimental.pallas import tpu as pltpu
from jax.experimental.pallas import tpu_sc as plsc

N = 10000
E = 320000
H = 128
L = 20
N_CLASS = 10
N_LAYER = 3


_TIL = 16
_EPT = E // _TIL
_CH = 512
_NCH = _EPT // _CH
_TAIL = _EPT - _NCH * _CH
_NHALF = 5120
_DUMP = _NHALF
_ROWS_T = _NHALF // _TIL


def _remap(buf, n, base_n):
    for v in range(n // 16):
        d = buf[pl.ds(v * 16, 16)] - base_n
        ok = (d >= 0) & (d < _NHALF)
        buf[pl.ds(v * 16, 16)] = jnp.where(ok, d, _DUMP)


def _segsum2_kernel(msg_hbm, sig_hbm, idx_hbm, z_hbm, num_hbm, den_hbm,
                    vals_v, i0, i1, i2, i3, it, accum):
    c = lax.axis_index("c")
    s = lax.axis_index("s")
    base_n = c * _NHALF
    idx_bufs = (i0, i1, i2, i3)

    def zero_accum():
        pltpu.sync_copy(z_hbm, accum.at[pl.ds(s * _ROWS_T, _ROWS_T)])

    def run(vhbm):
        base = s * _EPT

        def chunk(k, carry):
            cb = base + k * _CH
            pltpu.sync_copy(vhbm.at[pl.ds(cb, _CH)], vals_v)
            for j in range(4):
                pltpu.sync_copy(idx_hbm.at[pl.ds(cb + j * 128, 128)],
                                idx_bufs[j])
            for j in range(4):
                _remap(idx_bufs[j], 128, base_n)
                pltpu.sync_copy(vals_v.at[pl.ds(j * 128, 128)],
                                accum.at[idx_bufs[j]], add=True)
            return carry

        lax.fori_loop(0, _NCH, chunk, 0)
        tb = base + _NCH * _CH
        pltpu.sync_copy(vhbm.at[pl.ds(tb, _TAIL)], vals_v.at[pl.ds(0, _TAIL)])
        pltpu.sync_copy(idx_hbm.at[pl.ds(tb, _TAIL)], it)
        _remap(it, _TAIL, base_n)
        pltpu.sync_copy(vals_v.at[pl.ds(0, _TAIL)], accum.at[it], add=True)

    def copy_out(dst_hbm):
        rb = s * _ROWS_T
        pltpu.sync_copy(accum.at[pl.ds(rb, _ROWS_T)],
                        vals_v.at[pl.ds(0, _ROWS_T)])
        pltpu.sync_copy(vals_v.at[pl.ds(0, _ROWS_T)],
                        dst_hbm.at[pl.ds(base_n + rb, _ROWS_T)])

    zero_accum()
    plsc.subcore_barrier()
    run(msg_hbm)
    plsc.subcore_barrier()
    copy_out(num_hbm)
    plsc.subcore_barrier()
    zero_accum()
    plsc.subcore_barrier()
    run(sig_hbm)
    plsc.subcore_barrier()
    copy_out(den_hbm)


@jax.jit
def _segsum2(msg, sig, dst):
    mesh = plsc.VectorSubcoreMesh(core_axis_name="c", subcore_axis_name="s")
    zeros = jnp.zeros((_ROWS_T, H), jnp.float32)
    k = functools.partial(
        pl.kernel,
        mesh=mesh,
        out_type=(jax.ShapeDtypeStruct((2 * _NHALF, H), jnp.float32),
                  jax.ShapeDtypeStruct((2 * _NHALF, H), jnp.float32)),
        scratch_types=[
            pltpu.VMEM((_CH, H), jnp.float32),
            pltpu.VMEM((128,), jnp.int32),
            pltpu.VMEM((128,), jnp.int32),
            pltpu.VMEM((128,), jnp.int32),
            pltpu.VMEM((128,), jnp.int32),
            pltpu.VMEM((_TAIL,), jnp.int32),
            pltpu.VMEM_SHARED((_NHALF + 8, H), jnp.float32),
        ],
    )(_segsum2_kernel)
    num, den = k(msg, sig, dst, zeros)
    return num[:N], den[:N]


_GW = 32
_GEPW = E // _GW
_GCH = 256
_GNCH = _GEPW // _GCH
_GTAIL = _GEPW - _GNCH * _GCH


def _gather2_kernel(db_hbm, eh_hbm, src_hbm, dst_hbm,
                    dsbs_hbm, es_hbm,
                    dbuf, ebuf, s0, s1, d0, d1, st, dt):
    c = lax.axis_index("c")
    s = lax.axis_index("s")
    w = s * 2 + c
    base = w * _GEPW
    sbufs = (s0, s1)
    dbufs = (d0, d1)

    def chunk(k, carry):
        cb = base + k * _GCH
        for j in range(2):
            pltpu.sync_copy(src_hbm.at[pl.ds(cb + j * 128, 128)], sbufs[j])
            pltpu.sync_copy(dst_hbm.at[pl.ds(cb + j * 128, 128)], dbufs[j])
        for j in range(2):
            pltpu.sync_copy(db_hbm.at[sbufs[j]],
                            dbuf.at[pl.ds(j * 128, 128)])
            pltpu.sync_copy(eh_hbm.at[dbufs[j]],
                            ebuf.at[pl.ds(j * 128, 128)])
        pltpu.sync_copy(dbuf, dsbs_hbm.at[pl.ds(cb, _GCH)])
        pltpu.sync_copy(ebuf, es_hbm.at[pl.ds(cb, _GCH)])
        return carry

    lax.fori_loop(0, _GNCH, chunk, 0)
    tb = base + _GNCH * _GCH
    pltpu.sync_copy(src_hbm.at[pl.ds(tb, _GTAIL)], st)
    pltpu.sync_copy(dst_hbm.at[pl.ds(tb, _GTAIL)], dt)
    pltpu.sync_copy(db_hbm.at[st], dbuf.at[pl.ds(0, _GTAIL)])
    pltpu.sync_copy(eh_hbm.at[dt], ebuf.at[pl.ds(0, _GTAIL)])
    pltpu.sync_copy(dbuf.at[pl.ds(0, _GTAIL)], dsbs_hbm.at[pl.ds(tb, _GTAIL)])
    pltpu.sync_copy(ebuf.at[pl.ds(0, _GTAIL)], es_hbm.at[pl.ds(tb, _GTAIL)])


@jax.jit
def _gather2(dh_bh, eh, src, dst):
    mesh = plsc.VectorSubcoreMesh(core_axis_name="c", subcore_axis_name="s")
    k = functools.partial(
        pl.kernel,
        mesh=mesh,
        out_type=(jax.ShapeDtypeStruct((E, 2 * H), jnp.float32),
                  jax.ShapeDtypeStruct((E, H), jnp.float32)),
        scratch_types=[
            pltpu.VMEM((_GCH, 2 * H), jnp.float32),
            pltpu.VMEM((_GCH, H), jnp.float32),
            pltpu.VMEM((128,), jnp.int32),
            pltpu.VMEM((128,), jnp.int32),
            pltpu.VMEM((128,), jnp.int32),
            pltpu.VMEM((128,), jnp.int32),
            pltpu.VMEM((_GTAIL,), jnp.int32),
            pltpu.VMEM((_GTAIL,), jnp.int32),
        ],
    )(_gather2_kernel)
    return k(dh_bh, eh, src, dst)


_LB = 512
_NPAD = 10240


def _bilstm_kernel(x_ref, h_ref, len_ref,
                   wif_ref, whf_ref, bf_ref, wib_ref, whb_ref, bb_ref,
                   wh_ref, bh_ref, out_ref):
    x = x_ref[...]
    ln = len_ref[...]
    wif, whf, bf = wif_ref[...], whf_ref[...], bf_ref[...]
    wib, whb, bb = wib_ref[...], whb_ref[...], bb_ref[...]

    def step(xt, h, c, wi, wh, b):
        g = jnp.dot(xt, wi, preferred_element_type=jnp.float32)
        g = g + jnp.dot(h, wh, preferred_element_type=jnp.float32) + b
        i_ = jax.nn.sigmoid(g[:, 0:128])
        f_ = jax.nn.sigmoid(g[:, 128:256])
        g_ = jnp.tanh(g[:, 256:384])
        o_ = jax.nn.sigmoid(g[:, 384:512])
        c2 = f_ * c + i_ * g_
        h2 = o_ * jnp.tanh(c2)
        return h2, c2

    z = jnp.zeros((_LB, H), jnp.float32)
    h, c, hf = z, z, z
    for t in range(L):
        xt = x[:, t * H:(t + 1) * H]
        h, c = step(xt, h, c, wif, whf, bf)
        sel = (ln == t + 1).astype(jnp.float32)
        hf = hf + sel * h
    h, c = z, z
    for u in range(L - 1, -1, -1):
        xt = x[:, u * H:(u + 1) * H]
        h2, c2 = step(xt, h, c, wib, whb, bb)
        act = ln > u
        h = jnp.where(act, h2, h)
        c = jnp.where(act, c2, c)
    te = 0.5 * (hf + h)
    nrm = jnp.sqrt(jnp.sum(te * te, axis=1, keepdims=True))
    te = te / jnp.maximum(nrm, 1e-12)
    out_ref[...] = (
        jnp.dot(h_ref[...], wh_ref[...], preferred_element_type=jnp.float32)
        + bh_ref[...] + te)


@jax.jit
def _bilstm_hcur(txt2d, hpad, len2d, pf, pb, ph):
    grid = (_NPAD // _LB,)
    k = pl.pallas_call(
        _bilstm_kernel,
        grid=grid,
        in_specs=[
            pl.BlockSpec((_LB, L * H), lambda i: (i, 0)),
            pl.BlockSpec((_LB, H), lambda i: (i, 0)),
            pl.BlockSpec((_LB, 1), lambda i: (i, 0)),
            pl.BlockSpec((H, 4 * H), lambda i: (0, 0)),
            pl.BlockSpec((H, 4 * H), lambda i: (0, 0)),
            pl.BlockSpec((1, 4 * H), lambda i: (0, 0)),
            pl.BlockSpec((H, 4 * H), lambda i: (0, 0)),
            pl.BlockSpec((H, 4 * H), lambda i: (0, 0)),
            pl.BlockSpec((1, 4 * H), lambda i: (0, 0)),
            pl.BlockSpec((H, H), lambda i: (0, 0)),
            pl.BlockSpec((1, H), lambda i: (0, 0)),
        ],
        out_specs=pl.BlockSpec((_LB, H), lambda i: (i, 0)),
        out_shape=jax.ShapeDtypeStruct((_NPAD, H), jnp.float32),
    )
    return k(txt2d, hpad, len2d,
             pf["W_ih"].T, pf["W_hh"].T, pf["b"][None, :],
             pb["W_ih"].T, pb["W_hh"].T, pb["b"][None, :],
             ph["W"], ph["b"][None, :])


def _lin(p, x):
    return x @ p["W"] + p["b"]


def _bn(x, g, b):
    m = jnp.mean(x, axis=0)
    v = jnp.var(x, axis=0)
    return (x - m) / jnp.sqrt(v + 1e-5) * g + b


def _lstm(x, p):
    n = x.shape[0]
    xT = jnp.swapaxes(x, 0, 1)
    h0 = jnp.zeros((n, H), x.dtype)
    c0 = jnp.zeros((n, H), x.dtype)

    def step(carry, xt):
        hh, cc = carry
        gates = xt @ p["W_ih"].T + hh @ p["W_hh"].T + p["b"]
        i_, f_, g_, o_ = jnp.split(gates, 4, axis=-1)
        i_ = jax.nn.sigmoid(i_)
        f_ = jax.nn.sigmoid(f_)
        g_ = jnp.tanh(g_)
        o_ = jax.nn.sigmoid(o_)
        cc = f_ * cc + i_ * g_
        hh = o_ * jnp.tanh(cc)
        return (hh, cc), hh

    _, hs = jax.lax.scan(step, (h0, c0), xT)
    return jnp.swapaxes(hs, 0, 1)


def _mlp_head_kernel(y_ref, w0_ref, b0_ref, w1_ref, b1_ref, w2_ref, b2_ref,
                     out_ref):
    y = y_ref[...]
    y = jnp.maximum(y @ w0_ref[...] + b0_ref[...], 0.0)
    y = jnp.maximum(y @ w1_ref[...] + b1_ref[...], 0.0)
    out_ref[...] = y @ w2_ref[...] + b2_ref[...]


def _mlp_head(y, mlp):
    n = y.shape[0]
    blk = 2000
    grid = (n // blk,)
    w0, b0 = mlp[0]["W"], mlp[0]["b"]
    w1, b1 = mlp[1]["W"], mlp[1]["b"]
    w2, b2 = mlp[2]["W"], mlp[2]["b"]
    w2p = jnp.zeros((w2.shape[0], 128), w2.dtype).at[:, :N_CLASS].set(w2)
    b2p = jnp.zeros((128,), b2.dtype).at[:N_CLASS].set(b2)
    out = pl.pallas_call(
        _mlp_head_kernel,
        grid=grid,
        in_specs=[
            pl.BlockSpec((blk, H), lambda i: (i, 0)),
            pl.BlockSpec((H, 64), lambda i: (0, 0)),
            pl.BlockSpec((64,), lambda i: (0,)),
            pl.BlockSpec((64, 32), lambda i: (0, 0)),
            pl.BlockSpec((32,), lambda i: (0,)),
            pl.BlockSpec((32, 128), lambda i: (0, 0)),
            pl.BlockSpec((128,), lambda i: (0,)),
        ],
        out_specs=pl.BlockSpec((blk, 128), lambda i: (i, 0)),
        out_shape=jax.ShapeDtypeStruct((n, 128), y.dtype),
    )(y, w0, b0, w1, b1, w2p, b2p)
    return out[:, :N_CLASS]


def kernel(h, e, text, snorm_n, snorm_e, edge_index, text_length,
           graph_node_size, graph_edge_size, params):
    txt = jnp.take(params["text_emb"], text, axis=0)
    txt2d = jnp.zeros((_NPAD, L * H), jnp.float32).at[:N].set(
        txt.reshape(N, L * H))
    hpad = jnp.zeros((_NPAD, H), jnp.float32).at[:N].set(h)
    len2d = jnp.ones((_NPAD, 1), jnp.int32).at[:N, 0].set(text_length.astype(jnp.int32))
    hcur = _bilstm_hcur(txt2d, hpad, len2d,
                        params["lstm_f"], params["lstm_b"],
                        params["h_emb"])[:N]
    ecur = _lin(params["e_emb"], e)
    src = edge_index[0]
    dst = edge_index[1]
    n_nodes = h.shape[0]
    all_h = [hcur]
    for li in range(N_LAYER):
        lay = params["layers"][li]
        Ah = _lin(lay["A"], hcur)
        Bh = _lin(lay["B"], hcur)
        Ce = _lin(lay["C"], ecur)
        Dh = _lin(lay["D"], hcur)
        Eh = _lin(lay["E"], hcur)
        DSBS, ES = _gather2(jnp.concatenate([Dh, Bh], axis=1), Eh, src, dst)
        e_new = DSBS[:, :H] + ES + Ce
        sigma = jax.nn.sigmoid(e_new)
        num, den = _segsum2(sigma * DSBS[:, H:], sigma, dst)
        hn = Ah + num / (den + 1e-6)
        hn = _bn(hn * snorm_n, lay["bn_h_g"], lay["bn_h_b"])
        hn = hcur + jax.nn.relu(hn)
        en = _bn(e_new * snorm_e, lay["bn_e_g"], lay["bn_e_b"])
        en = ecur + jax.nn.relu(en)
        all_h.append(hn)
        hcur = jax.nn.relu(_lin(params["dense"][li], jnp.concatenate(all_h, axis=1)))
        ecur = en
    return _mlp_head(hcur, params["mlp"])


# speedup vs baseline: 2.7428x; 1.1508x over previous
import functools

import jax
import jax.numpy as jnp
from jax import lax
from jax.experimental import pallas as pl
from jax.experimental.pallas import tpu as pltpu
from jax.experimental.pallas import tpu_sc as plsc

N = 10000
E = 320000
H = 128
L = 20
N_CLASS = 10
N_LAYER = 3


_TIL = 16
_EPT = E // _TIL
_CH = 512
_NCH = _EPT // _CH
_TAIL = _EPT - _NCH * _CH
_NHALF = 5120
_DUMP = _NHALF
_ROWS_T = _NHALF // _TIL


def _remap(buf, n, base_n):
    for v in range(n // 16):
        d = buf[pl.ds(v * 16, 16)] - base_n
        ok = (d >= 0) & (d < _NHALF)
        buf[pl.ds(v * 16, 16)] = jnp.where(ok, d, _DUMP)


def _segsum2_kernel(msg_hbm, sig_hbm, idx_hbm, z_hbm, num_hbm, den_hbm,
                    vals_v, i0, i1, i2, i3, it, accum, sem):
    c = lax.axis_index("c")
    s = lax.axis_index("s")
    base_n = c * _NHALF
    idx_bufs = (i0, i1, i2, i3)

    def zero_accum():
        pltpu.sync_copy(z_hbm, accum.at[pl.ds(s * _ROWS_T, _ROWS_T)])

    def run(vhbm):
        base = s * _EPT

        def chunk(k, carry):
            cb = base + k * _CH
            ds_ = [pltpu.async_copy(vhbm.at[pl.ds(cb, _CH)], vals_v, sem)]
            for j in range(4):
                ds_.append(pltpu.async_copy(
                    idx_hbm.at[pl.ds(cb + j * 128, 128)], idx_bufs[j], sem))
            for d in ds_:
                d.wait()
            for j in range(4):
                _remap(idx_bufs[j], 128, base_n)
            sc = [pltpu.async_copy(vals_v.at[pl.ds(j * 128, 128)],
                                   accum.at[idx_bufs[j]], sem, add=True)
                  for j in range(4)]
            for d in sc:
                d.wait()
            return carry

        lax.fori_loop(0, _NCH, chunk, 0)
        tb = base + _NCH * _CH
        pltpu.sync_copy(vhbm.at[pl.ds(tb, _TAIL)], vals_v.at[pl.ds(0, _TAIL)])
        pltpu.sync_copy(idx_hbm.at[pl.ds(tb, _TAIL)], it)
        _remap(it, _TAIL, base_n)
        pltpu.sync_copy(vals_v.at[pl.ds(0, _TAIL)], accum.at[it], add=True)

    def copy_out(dst_hbm):
        rb = s * _ROWS_T
        pltpu.sync_copy(accum.at[pl.ds(rb, _ROWS_T)],
                        vals_v.at[pl.ds(0, _ROWS_T)])
        pltpu.sync_copy(vals_v.at[pl.ds(0, _ROWS_T)],
                        dst_hbm.at[pl.ds(base_n + rb, _ROWS_T)])

    zero_accum()
    plsc.subcore_barrier()
    run(msg_hbm)
    plsc.subcore_barrier()
    copy_out(num_hbm)
    plsc.subcore_barrier()
    zero_accum()
    plsc.subcore_barrier()
    run(sig_hbm)
    plsc.subcore_barrier()
    copy_out(den_hbm)


@jax.jit
def _segsum2(msg, sig, dst):
    mesh = plsc.VectorSubcoreMesh(core_axis_name="c", subcore_axis_name="s")
    zeros = jnp.zeros((_ROWS_T, H), jnp.float32)
    k = functools.partial(
        pl.kernel,
        mesh=mesh,
        out_type=(jax.ShapeDtypeStruct((2 * _NHALF, H), jnp.float32),
                  jax.ShapeDtypeStruct((2 * _NHALF, H), jnp.float32)),
        scratch_types=[
            pltpu.VMEM((_CH, H), jnp.float32),
            pltpu.VMEM((128,), jnp.int32),
            pltpu.VMEM((128,), jnp.int32),
            pltpu.VMEM((128,), jnp.int32),
            pltpu.VMEM((128,), jnp.int32),
            pltpu.VMEM((_TAIL,), jnp.int32),
            pltpu.VMEM_SHARED((_NHALF + 8, H), jnp.float32),
            pltpu.SemaphoreType.DMA,
        ],
    )(_segsum2_kernel)
    num, den = k(msg, sig, dst, zeros)
    return num[:N], den[:N]


_GW = 32
_GEPW = E // _GW
_GCH = 256
_GNCH = _GEPW // _GCH
_GTAIL = _GEPW - _GNCH * _GCH


def _gather2_kernel(db_hbm, eh_hbm, src_hbm, dst_hbm,
                    dsbs_hbm, es_hbm,
                    dbuf, ebuf, s0, s1, d0, d1, st, dt, sem):
    c = lax.axis_index("c")
    s = lax.axis_index("s")
    w = s * 2 + c
    base = w * _GEPW
    sbufs = (s0, s1)
    dbufs = (d0, d1)

    def chunk(k, carry):
        cb = base + k * _GCH
        ds_ = []
        for j in range(2):
            ds_.append(pltpu.async_copy(
                src_hbm.at[pl.ds(cb + j * 128, 128)], sbufs[j], sem))
            ds_.append(pltpu.async_copy(
                dst_hbm.at[pl.ds(cb + j * 128, 128)], dbufs[j], sem))
        for d in ds_:
            d.wait()
        gs = []
        for j in range(2):
            gs.append(pltpu.async_copy(
                db_hbm.at[sbufs[j]], dbuf.at[pl.ds(j * 128, 128)], sem))
            gs.append(pltpu.async_copy(
                eh_hbm.at[dbufs[j]], ebuf.at[pl.ds(j * 128, 128)], sem))
        for d in gs:
            d.wait()
        ws = [pltpu.async_copy(dbuf, dsbs_hbm.at[pl.ds(cb, _GCH)], sem),
              pltpu.async_copy(ebuf, es_hbm.at[pl.ds(cb, _GCH)], sem)]
        for d in ws:
            d.wait()
        return carry

    lax.fori_loop(0, _GNCH, chunk, 0)
    tb = base + _GNCH * _GCH
    pltpu.sync_copy(src_hbm.at[pl.ds(tb, _GTAIL)], st)
    pltpu.sync_copy(dst_hbm.at[pl.ds(tb, _GTAIL)], dt)
    pltpu.sync_copy(db_hbm.at[st], dbuf.at[pl.ds(0, _GTAIL)])
    pltpu.sync_copy(eh_hbm.at[dt], ebuf.at[pl.ds(0, _GTAIL)])
    pltpu.sync_copy(dbuf.at[pl.ds(0, _GTAIL)], dsbs_hbm.at[pl.ds(tb, _GTAIL)])
    pltpu.sync_copy(ebuf.at[pl.ds(0, _GTAIL)], es_hbm.at[pl.ds(tb, _GTAIL)])


@jax.jit
def _gather2(dh_bh, eh, src, dst):
    mesh = plsc.VectorSubcoreMesh(core_axis_name="c", subcore_axis_name="s")
    k = functools.partial(
        pl.kernel,
        mesh=mesh,
        out_type=(jax.ShapeDtypeStruct((E, 2 * H), jnp.float32),
                  jax.ShapeDtypeStruct((E, H), jnp.float32)),
        scratch_types=[
            pltpu.VMEM((_GCH, 2 * H), jnp.float32),
            pltpu.VMEM((_GCH, H), jnp.float32),
            pltpu.VMEM((128,), jnp.int32),
            pltpu.VMEM((128,), jnp.int32),
            pltpu.VMEM((128,), jnp.int32),
            pltpu.VMEM((128,), jnp.int32),
            pltpu.VMEM((_GTAIL,), jnp.int32),
            pltpu.VMEM((_GTAIL,), jnp.int32),
            pltpu.SemaphoreType.DMA,
        ],
    )(_gather2_kernel)
    return k(dh_bh, eh, src, dst)


_LB = 512
_NPAD = 10240


def _bilstm_kernel(x_ref, h_ref, len_ref,
                   wif_ref, whf_ref, bf_ref, wib_ref, whb_ref, bb_ref,
                   wh_ref, bh_ref, out_ref):
    x = x_ref[...]
    ln = len_ref[...]
    wif, whf, bf = wif_ref[...], whf_ref[...], bf_ref[...]
    wib, whb, bb = wib_ref[...], whb_ref[...], bb_ref[...]

    def step(xt, h, c, wi, wh, b):
        g = jnp.dot(xt, wi, preferred_element_type=jnp.float32)
        g = g + jnp.dot(h, wh, preferred_element_type=jnp.float32) + b
        i_ = jax.nn.sigmoid(g[:, 0:128])
        f_ = jax.nn.sigmoid(g[:, 128:256])
        g_ = jnp.tanh(g[:, 256:384])
        o_ = jax.nn.sigmoid(g[:, 384:512])
        c2 = f_ * c + i_ * g_
        h2 = o_ * jnp.tanh(c2)
        return h2, c2

    z = jnp.zeros((_LB, H), jnp.float32)
    h, c, hf = z, z, z
    for t in range(L):
        xt = x[:, t * H:(t + 1) * H]
        h, c = step(xt, h, c, wif, whf, bf)
        sel = (ln == t + 1).astype(jnp.float32)
        hf = hf + sel * h
    h, c = z, z
    for u in range(L - 1, -1, -1):
        xt = x[:, u * H:(u + 1) * H]
        h2, c2 = step(xt, h, c, wib, whb, bb)
        act = ln > u
        h = jnp.where(act, h2, h)
        c = jnp.where(act, c2, c)
    te = 0.5 * (hf + h)
    nrm = jnp.sqrt(jnp.sum(te * te, axis=1, keepdims=True))
    te = te / jnp.maximum(nrm, 1e-12)
    out_ref[...] = (
        jnp.dot(h_ref[...], wh_ref[...], preferred_element_type=jnp.float32)
        + bh_ref[...] + te)


@jax.jit
def _bilstm_hcur(txt2d, hpad, len2d, pf, pb, ph):
    grid = (_NPAD // _LB,)
    k = pl.pallas_call(
        _bilstm_kernel,
        grid=grid,
        in_specs=[
            pl.BlockSpec((_LB, L * H), lambda i: (i, 0)),
            pl.BlockSpec((_LB, H), lambda i: (i, 0)),
            pl.BlockSpec((_LB, 1), lambda i: (i, 0)),
            pl.BlockSpec((H, 4 * H), lambda i: (0, 0)),
            pl.BlockSpec((H, 4 * H), lambda i: (0, 0)),
            pl.BlockSpec((1, 4 * H), lambda i: (0, 0)),
            pl.BlockSpec((H, 4 * H), lambda i: (0, 0)),
            pl.BlockSpec((H, 4 * H), lambda i: (0, 0)),
            pl.BlockSpec((1, 4 * H), lambda i: (0, 0)),
            pl.BlockSpec((H, H), lambda i: (0, 0)),
            pl.BlockSpec((1, H), lambda i: (0, 0)),
        ],
        out_specs=pl.BlockSpec((_LB, H), lambda i: (i, 0)),
        out_shape=jax.ShapeDtypeStruct((_NPAD, H), jnp.float32),
    )
    return k(txt2d, hpad, len2d,
             pf["W_ih"].T, pf["W_hh"].T, pf["b"][None, :],
             pb["W_ih"].T, pb["W_hh"].T, pb["b"][None, :],
             ph["W"], ph["b"][None, :])


def _lin(p, x):
    return x @ p["W"] + p["b"]


def _bn(x, g, b):
    m = jnp.mean(x, axis=0)
    v = jnp.var(x, axis=0)
    return (x - m) / jnp.sqrt(v + 1e-5) * g + b


def _lstm(x, p):
    n = x.shape[0]
    xT = jnp.swapaxes(x, 0, 1)
    h0 = jnp.zeros((n, H), x.dtype)
    c0 = jnp.zeros((n, H), x.dtype)

    def step(carry, xt):
        hh, cc = carry
        gates = xt @ p["W_ih"].T + hh @ p["W_hh"].T + p["b"]
        i_, f_, g_, o_ = jnp.split(gates, 4, axis=-1)
        i_ = jax.nn.sigmoid(i_)
        f_ = jax.nn.sigmoid(f_)
        g_ = jnp.tanh(g_)
        o_ = jax.nn.sigmoid(o_)
        cc = f_ * cc + i_ * g_
        hh = o_ * jnp.tanh(cc)
        return (hh, cc), hh

    _, hs = jax.lax.scan(step, (h0, c0), xT)
    return jnp.swapaxes(hs, 0, 1)


def _mlp_head_kernel(y_ref, w0_ref, b0_ref, w1_ref, b1_ref, w2_ref, b2_ref,
                     out_ref):
    y = y_ref[...]
    y = jnp.maximum(y @ w0_ref[...] + b0_ref[...], 0.0)
    y = jnp.maximum(y @ w1_ref[...] + b1_ref[...], 0.0)
    out_ref[...] = y @ w2_ref[...] + b2_ref[...]


def _mlp_head(y, mlp):
    n = y.shape[0]
    blk = 2000
    grid = (n // blk,)
    w0, b0 = mlp[0]["W"], mlp[0]["b"]
    w1, b1 = mlp[1]["W"], mlp[1]["b"]
    w2, b2 = mlp[2]["W"], mlp[2]["b"]
    w2p = jnp.zeros((w2.shape[0], 128), w2.dtype).at[:, :N_CLASS].set(w2)
    b2p = jnp.zeros((128,), b2.dtype).at[:N_CLASS].set(b2)
    out = pl.pallas_call(
        _mlp_head_kernel,
        grid=grid,
        in_specs=[
            pl.BlockSpec((blk, H), lambda i: (i, 0)),
            pl.BlockSpec((H, 64), lambda i: (0, 0)),
            pl.BlockSpec((64,), lambda i: (0,)),
            pl.BlockSpec((64, 32), lambda i: (0, 0)),
            pl.BlockSpec((32,), lambda i: (0,)),
            pl.BlockSpec((32, 128), lambda i: (0, 0)),
            pl.BlockSpec((128,), lambda i: (0,)),
        ],
        out_specs=pl.BlockSpec((blk, 128), lambda i: (i, 0)),
        out_shape=jax.ShapeDtypeStruct((n, 128), y.dtype),
    )(y, w0, b0, w1, b1, w2p, b2p)
    return out[:, :N_CLASS]


def kernel(h, e, text, snorm_n, snorm_e, edge_index, text_length,
           graph_node_size, graph_edge_size, params):
    txt = jnp.take(params["text_emb"], text, axis=0)
    txt2d = jnp.zeros((_NPAD, L * H), jnp.float32).at[:N].set(
        txt.reshape(N, L * H))
    hpad = jnp.zeros((_NPAD, H), jnp.float32).at[:N].set(h)
    len2d = jnp.ones((_NPAD, 1), jnp.int32).at[:N, 0].set(text_length.astype(jnp.int32))
    hcur = _bilstm_hcur(txt2d, hpad, len2d,
                        params["lstm_f"], params["lstm_b"],
                        params["h_emb"])[:N]
    ecur = _lin(params["e_emb"], e)
    src = edge_index[0]
    dst = edge_index[1]
    n_nodes = h.shape[0]
    all_h = [hcur]
    for li in range(N_LAYER):
        lay = params["layers"][li]
        Ah = _lin(lay["A"], hcur)
        Bh = _lin(lay["B"], hcur)
        Ce = _lin(lay["C"], ecur)
        Dh = _lin(lay["D"], hcur)
        Eh = _lin(lay["E"], hcur)
        DSBS, ES = _gather2(jnp.concatenate([Dh, Bh], axis=1), Eh, src, dst)
        e_new = DSBS[:, :H] + ES + Ce
        sigma = jax.nn.sigmoid(e_new)
        num, den = _segsum2(sigma * DSBS[:, H:], sigma, dst)
        hn = Ah + num / (den + 1e-6)
        hn = _bn(hn * snorm_n, lay["bn_h_g"], lay["bn_h_b"])
        hn = hcur + jax.nn.relu(hn)
        en = _bn(e_new * snorm_e, lay["bn_e_g"], lay["bn_e_b"])
        en = ecur + jax.nn.relu(en)
        all_h.append(hn)
        hcur = jax.nn.relu(_lin(params["dense"][li], jnp.concatenate(all_h, axis=1)))
        ecur = en
    return _mlp_head(hcur, params["mlp"])


# speedup vs baseline: 2.8096x; 1.0243x over previous
import functools

import jax
import jax.numpy as jnp
from jax import lax
from jax.experimental import pallas as pl
from jax.experimental.pallas import tpu as pltpu
from jax.experimental.pallas import tpu_sc as plsc

N = 10000
E = 320000
H = 128
L = 20
N_CLASS = 10
N_LAYER = 3


_TIL = 16
_EPT = E // _TIL
_CH = 512
_NCH = _EPT // _CH
_TAIL = _EPT - _NCH * _CH
_NHALF = 5120
_DUMP = _NHALF
_ROWS_T = _NHALF // _TIL


def _remap(buf, n, base_n):
    for v in range(n // 16):
        d = buf[pl.ds(v * 16, 16)] - base_n
        ok = (d >= 0) & (d < _NHALF)
        buf[pl.ds(v * 16, 16)] = jnp.where(ok, d, _DUMP)


def _segsum2_kernel(msg_hbm, sig_hbm, idx_hbm, z_hbm, num_hbm, den_hbm,
                    vals_v, i0, i1, i2, i3, it, accum, sem):
    c = lax.axis_index("c")
    s = lax.axis_index("s")
    base_n = c * _NHALF
    idx_bufs = (i0, i1, i2, i3)

    def zero_accum():
        pltpu.sync_copy(z_hbm, accum.at[pl.ds(s * _ROWS_T, _ROWS_T)])

    def run(vhbm):
        base = s * _EPT

        def chunk(k, carry):
            cb = base + k * _CH
            ds_ = [pltpu.async_copy(vhbm.at[pl.ds(cb, _CH)], vals_v, sem)]
            for j in range(4):
                ds_.append(pltpu.async_copy(
                    idx_hbm.at[pl.ds(cb + j * 128, 128)], idx_bufs[j], sem))
            for d in ds_:
                d.wait()
            for j in range(4):
                _remap(idx_bufs[j], 128, base_n)
            sc = [pltpu.async_copy(vals_v.at[pl.ds(j * 128, 128)],
                                   accum.at[idx_bufs[j]], sem, add=True)
                  for j in range(4)]
            for d in sc:
                d.wait()
            return carry

        lax.fori_loop(0, _NCH, chunk, 0)
        tb = base + _NCH * _CH
        pltpu.sync_copy(vhbm.at[pl.ds(tb, _TAIL)], vals_v.at[pl.ds(0, _TAIL)])
        pltpu.sync_copy(idx_hbm.at[pl.ds(tb, _TAIL)], it)
        _remap(it, _TAIL, base_n)
        pltpu.sync_copy(vals_v.at[pl.ds(0, _TAIL)], accum.at[it], add=True)

    def copy_out(dst_hbm):
        rb = s * _ROWS_T
        pltpu.sync_copy(accum.at[pl.ds(rb, _ROWS_T)],
                        vals_v.at[pl.ds(0, _ROWS_T)])
        pltpu.sync_copy(vals_v.at[pl.ds(0, _ROWS_T)],
                        dst_hbm.at[pl.ds(base_n + rb, _ROWS_T)])

    zero_accum()
    plsc.subcore_barrier()
    run(msg_hbm)
    plsc.subcore_barrier()
    copy_out(num_hbm)
    plsc.subcore_barrier()
    zero_accum()
    plsc.subcore_barrier()
    run(sig_hbm)
    plsc.subcore_barrier()
    copy_out(den_hbm)


@jax.jit
def _segsum2(msg, sig, dst):
    mesh = plsc.VectorSubcoreMesh(core_axis_name="c", subcore_axis_name="s")
    zeros = jnp.zeros((_ROWS_T, H), jnp.float32)
    k = functools.partial(
        pl.kernel,
        mesh=mesh,
        out_type=(jax.ShapeDtypeStruct((2 * _NHALF, H), jnp.float32),
                  jax.ShapeDtypeStruct((2 * _NHALF, H), jnp.float32)),
        scratch_types=[
            pltpu.VMEM((_CH, H), jnp.float32),
            pltpu.VMEM((128,), jnp.int32),
            pltpu.VMEM((128,), jnp.int32),
            pltpu.VMEM((128,), jnp.int32),
            pltpu.VMEM((128,), jnp.int32),
            pltpu.VMEM((_TAIL,), jnp.int32),
            pltpu.VMEM_SHARED((_NHALF + 8, H), jnp.float32),
            pltpu.SemaphoreType.DMA,
        ],
    )(_segsum2_kernel)
    num, den = k(msg, sig, dst, zeros)
    return num[:N], den[:N]


_GW = 32
_GEPW = E // _GW
_GCH = 256
_GNCH = _GEPW // _GCH
_GTAIL = _GEPW - _GNCH * _GCH


def _gather2_kernel(db_hbm, eh_hbm, src_hbm, dst_hbm,
                    dsbs_hbm, es_hbm,
                    dbuf, ebuf, s0, s1, d0, d1, st, dt, sem):
    c = lax.axis_index("c")
    s = lax.axis_index("s")
    w = s * 2 + c
    base = w * _GEPW
    sbufs = (s0, s1)
    dbufs = (d0, d1)

    def chunk(k, carry):
        cb = base + k * _GCH
        ds_ = []
        for j in range(2):
            ds_.append(pltpu.async_copy(
                src_hbm.at[pl.ds(cb + j * 128, 128)], sbufs[j], sem))
            ds_.append(pltpu.async_copy(
                dst_hbm.at[pl.ds(cb + j * 128, 128)], dbufs[j], sem))
        for d in ds_:
            d.wait()
        gs = []
        for j in range(2):
            gs.append(pltpu.async_copy(
                db_hbm.at[sbufs[j]], dbuf.at[pl.ds(j * 128, 128)], sem))
            gs.append(pltpu.async_copy(
                eh_hbm.at[dbufs[j]], ebuf.at[pl.ds(j * 128, 128)], sem))
        for d in gs:
            d.wait()
        ws = [pltpu.async_copy(dbuf, dsbs_hbm.at[pl.ds(cb, _GCH)], sem),
              pltpu.async_copy(ebuf, es_hbm.at[pl.ds(cb, _GCH)], sem)]
        for d in ws:
            d.wait()
        return carry

    lax.fori_loop(0, _GNCH, chunk, 0)
    tb = base + _GNCH * _GCH
    pltpu.sync_copy(src_hbm.at[pl.ds(tb, _GTAIL)], st)
    pltpu.sync_copy(dst_hbm.at[pl.ds(tb, _GTAIL)], dt)
    pltpu.sync_copy(db_hbm.at[st], dbuf.at[pl.ds(0, _GTAIL)])
    pltpu.sync_copy(eh_hbm.at[dt], ebuf.at[pl.ds(0, _GTAIL)])
    pltpu.sync_copy(dbuf.at[pl.ds(0, _GTAIL)], dsbs_hbm.at[pl.ds(tb, _GTAIL)])
    pltpu.sync_copy(ebuf.at[pl.ds(0, _GTAIL)], es_hbm.at[pl.ds(tb, _GTAIL)])


@jax.jit
def _gather2(dh_bh, eh, src, dst):
    mesh = plsc.VectorSubcoreMesh(core_axis_name="c", subcore_axis_name="s")
    k = functools.partial(
        pl.kernel,
        mesh=mesh,
        out_type=(jax.ShapeDtypeStruct((E, 2 * H), jnp.float32),
                  jax.ShapeDtypeStruct((E, H), jnp.float32)),
        scratch_types=[
            pltpu.VMEM((_GCH, 2 * H), jnp.float32),
            pltpu.VMEM((_GCH, H), jnp.float32),
            pltpu.VMEM((128,), jnp.int32),
            pltpu.VMEM((128,), jnp.int32),
            pltpu.VMEM((128,), jnp.int32),
            pltpu.VMEM((128,), jnp.int32),
            pltpu.VMEM((_GTAIL,), jnp.int32),
            pltpu.VMEM((_GTAIL,), jnp.int32),
            pltpu.SemaphoreType.DMA,
        ],
    )(_gather2_kernel)
    return k(dh_bh, eh, src, dst)


_TGW = 32
_TIDS = 204800
_TPW = _TIDS // _TGW
_TCH = 256
_TNCH = _TPW // _TCH


def _tgather_kernel(tab_hbm, ids_hbm, out_hbm, rbuf, i0, i1, sem):
    c = lax.axis_index("c")
    s = lax.axis_index("s")
    base = (s * 2 + c) * _TPW
    ibufs = (i0, i1)

    def chunk(k, carry):
        cb = base + k * _TCH
        ds_ = [pltpu.async_copy(ids_hbm.at[pl.ds(cb + j * 128, 128)],
                                ibufs[j], sem) for j in range(2)]
        for d in ds_:
            d.wait()
        gs = [pltpu.async_copy(tab_hbm.at[ibufs[j]],
                               rbuf.at[pl.ds(j * 128, 128)], sem)
              for j in range(2)]
        for d in gs:
            d.wait()
        pltpu.async_copy(rbuf, out_hbm.at[pl.ds(cb, _TCH)], sem).wait()
        return carry

    lax.fori_loop(0, _TNCH, chunk, 0)


@jax.jit
def _tgather(tab, ids):
    mesh = plsc.VectorSubcoreMesh(core_axis_name="c", subcore_axis_name="s")
    k = functools.partial(
        pl.kernel,
        mesh=mesh,
        out_type=jax.ShapeDtypeStruct((_TIDS, H), jnp.float32),
        scratch_types=[
            pltpu.VMEM((_TCH, H), jnp.float32),
            pltpu.VMEM((128,), jnp.int32),
            pltpu.VMEM((128,), jnp.int32),
            pltpu.SemaphoreType.DMA,
        ],
    )(_tgather_kernel)
    return k(tab, ids)


_EB = 2000
_ENB = E // _EB


def _edge1_kernel(dsbs_ref, es_ref, ec_ref, c_ref, bc_ref, sn_ref,
                  enew_ref, sig_ref, msg_ref, st_ref, acc):
    i = pl.program_id(0)

    @pl.when(i == 0)
    def _():
        acc[...] = jnp.zeros_like(acc)

    d = dsbs_ref[...]
    x = d[:, :H] + es_ref[...] + bc_ref[...]
    x = x + jnp.dot(ec_ref[...], c_ref[...], preferred_element_type=jnp.float32)
    sg = jax.nn.sigmoid(x)
    enew_ref[...] = x
    sig_ref[...] = sg
    msg_ref[...] = sg * d[:, H:]
    y = x * sn_ref[...]
    acc[0:1, :] += jnp.sum(y, axis=0, keepdims=True)
    acc[1:2, :] += jnp.sum(y * y, axis=0, keepdims=True)

    @pl.when(i == _ENB - 1)
    def _():
        st_ref[...] = acc[...]


@jax.jit
def _edge_pass1(dsbs, es, ecur, cw, cb, sn):
    sh = jax.ShapeDtypeStruct((E, H), jnp.float32)
    out = pl.pallas_call(
        _edge1_kernel,
        grid=(_ENB,),
        in_specs=[
            pl.BlockSpec((_EB, 2 * H), lambda i: (i, 0)),
            pl.BlockSpec((_EB, H), lambda i: (i, 0)),
            pl.BlockSpec((_EB, H), lambda i: (i, 0)),
            pl.BlockSpec((H, H), lambda i: (0, 0)),
            pl.BlockSpec((1, H), lambda i: (0, 0)),
            pl.BlockSpec((_EB, 1), lambda i: (i, 0)),
        ],
        out_specs=[
            pl.BlockSpec((_EB, H), lambda i: (i, 0)),
            pl.BlockSpec((_EB, H), lambda i: (i, 0)),
            pl.BlockSpec((_EB, H), lambda i: (i, 0)),
            pl.BlockSpec((8, H), lambda i: (0, 0)),
        ],
        out_shape=[sh, sh, sh, jax.ShapeDtypeStruct((8, H), jnp.float32)],
        scratch_shapes=[pltpu.VMEM((8, H), jnp.float32)],
    )(dsbs, es, ecur, cw, cb, sn)
    return out


_LB = 512
_NPAD = 10240


def _bilstm_kernel(x_ref, h_ref, len_ref,
                   wif_ref, whf_ref, bf_ref, wib_ref, whb_ref, bb_ref,
                   wh_ref, bh_ref, out_ref):
    x = x_ref[...]
    ln = len_ref[...]
    wif, whf, bf = wif_ref[...], whf_ref[...], bf_ref[...]
    wib, whb, bb = wib_ref[...], whb_ref[...], bb_ref[...]

    def step(xt, h, c, wi, wh, b):
        g = jnp.dot(xt, wi, preferred_element_type=jnp.float32)
        g = g + jnp.dot(h, wh, preferred_element_type=jnp.float32) + b
        i_ = jax.nn.sigmoid(g[:, 0:128])
        f_ = jax.nn.sigmoid(g[:, 128:256])
        g_ = jnp.tanh(g[:, 256:384])
        o_ = jax.nn.sigmoid(g[:, 384:512])
        c2 = f_ * c + i_ * g_
        h2 = o_ * jnp.tanh(c2)
        return h2, c2

    z = jnp.zeros((_LB, H), jnp.float32)
    h, c, hf = z, z, z
    for t in range(L):
        xt = x[:, t * H:(t + 1) * H]
        h, c = step(xt, h, c, wif, whf, bf)
        sel = (ln == t + 1).astype(jnp.float32)
        hf = hf + sel * h
    h, c = z, z
    for u in range(L - 1, -1, -1):
        xt = x[:, u * H:(u + 1) * H]
        h2, c2 = step(xt, h, c, wib, whb, bb)
        act = ln > u
        h = jnp.where(act, h2, h)
        c = jnp.where(act, c2, c)
    te = 0.5 * (hf + h)
    nrm = jnp.sqrt(jnp.sum(te * te, axis=1, keepdims=True))
    te = te / jnp.maximum(nrm, 1e-12)
    out_ref[...] = (
        jnp.dot(h_ref[...], wh_ref[...], preferred_element_type=jnp.float32)
        + bh_ref[...] + te)


@jax.jit
def _bilstm_hcur(txt2d, hpad, len2d, pf, pb, ph):
    grid = (_NPAD // _LB,)
    k = pl.pallas_call(
        _bilstm_kernel,
        grid=grid,
        in_specs=[
            pl.BlockSpec((_LB, L * H), lambda i: (i, 0)),
            pl.BlockSpec((_LB, H), lambda i: (i, 0)),
            pl.BlockSpec((_LB, 1), lambda i: (i, 0)),
            pl.BlockSpec((H, 4 * H), lambda i: (0, 0)),
            pl.BlockSpec((H, 4 * H), lambda i: (0, 0)),
            pl.BlockSpec((1, 4 * H), lambda i: (0, 0)),
            pl.BlockSpec((H, 4 * H), lambda i: (0, 0)),
            pl.BlockSpec((H, 4 * H), lambda i: (0, 0)),
            pl.BlockSpec((1, 4 * H), lambda i: (0, 0)),
            pl.BlockSpec((H, H), lambda i: (0, 0)),
            pl.BlockSpec((1, H), lambda i: (0, 0)),
        ],
        out_specs=pl.BlockSpec((_LB, H), lambda i: (i, 0)),
        out_shape=jax.ShapeDtypeStruct((_NPAD, H), jnp.float32),
    )
    return k(txt2d, hpad, len2d,
             pf["W_ih"].T, pf["W_hh"].T, pf["b"][None, :],
             pb["W_ih"].T, pb["W_hh"].T, pb["b"][None, :],
             ph["W"], ph["b"][None, :])


def _lin(p, x):
    return x @ p["W"] + p["b"]


def _bn(x, g, b):
    m = jnp.mean(x, axis=0)
    v = jnp.var(x, axis=0)
    return (x - m) / jnp.sqrt(v + 1e-5) * g + b


def _lstm(x, p):
    n = x.shape[0]
    xT = jnp.swapaxes(x, 0, 1)
    h0 = jnp.zeros((n, H), x.dtype)
    c0 = jnp.zeros((n, H), x.dtype)

    def step(carry, xt):
        hh, cc = carry
        gates = xt @ p["W_ih"].T + hh @ p["W_hh"].T + p["b"]
        i_, f_, g_, o_ = jnp.split(gates, 4, axis=-1)
        i_ = jax.nn.sigmoid(i_)
        f_ = jax.nn.sigmoid(f_)
        g_ = jnp.tanh(g_)
        o_ = jax.nn.sigmoid(o_)
        cc = f_ * cc + i_ * g_
        hh = o_ * jnp.tanh(cc)
        return (hh, cc), hh

    _, hs = jax.lax.scan(step, (h0, c0), xT)
    return jnp.swapaxes(hs, 0, 1)


def _mlp_head_kernel(y_ref, w0_ref, b0_ref, w1_ref, b1_ref, w2_ref, b2_ref,
                     out_ref):
    y = y_ref[...]
    y = jnp.maximum(y @ w0_ref[...] + b0_ref[...], 0.0)
    y = jnp.maximum(y @ w1_ref[...] + b1_ref[...], 0.0)
    out_ref[...] = y @ w2_ref[...] + b2_ref[...]


def _mlp_head(y, mlp):
    n = y.shape[0]
    blk = 2000
    grid = (n // blk,)
    w0, b0 = mlp[0]["W"], mlp[0]["b"]
    w1, b1 = mlp[1]["W"], mlp[1]["b"]
    w2, b2 = mlp[2]["W"], mlp[2]["b"]
    w2p = jnp.zeros((w2.shape[0], 128), w2.dtype).at[:, :N_CLASS].set(w2)
    b2p = jnp.zeros((128,), b2.dtype).at[:N_CLASS].set(b2)
    out = pl.pallas_call(
        _mlp_head_kernel,
        grid=grid,
        in_specs=[
            pl.BlockSpec((blk, H), lambda i: (i, 0)),
            pl.BlockSpec((H, 64), lambda i: (0, 0)),
            pl.BlockSpec((64,), lambda i: (0,)),
            pl.BlockSpec((64, 32), lambda i: (0, 0)),
            pl.BlockSpec((32,), lambda i: (0,)),
            pl.BlockSpec((32, 128), lambda i: (0, 0)),
            pl.BlockSpec((128,), lambda i: (0,)),
        ],
        out_specs=pl.BlockSpec((blk, 128), lambda i: (i, 0)),
        out_shape=jax.ShapeDtypeStruct((n, 128), y.dtype),
    )(y, w0, b0, w1, b1, w2p, b2p)
    return out[:, :N_CLASS]


def kernel(h, e, text, snorm_n, snorm_e, edge_index, text_length,
           graph_node_size, graph_edge_size, params):
    ids = jnp.zeros((_TIDS,), jnp.int32).at[:N * L].set(
        text.reshape(-1).astype(jnp.int32))
    txt2d = _tgather(params["text_emb"], ids).reshape(_NPAD, L * H)
    hpad = jnp.zeros((_NPAD, H), jnp.float32).at[:N].set(h)
    len2d = jnp.ones((_NPAD, 1), jnp.int32).at[:N, 0].set(text_length.astype(jnp.int32))
    hcur = _bilstm_hcur(txt2d, hpad, len2d,
                        params["lstm_f"], params["lstm_b"],
                        params["h_emb"])[:N]
    ecur = _lin(params["e_emb"], e)
    src = edge_index[0]
    dst = edge_index[1]
    n_nodes = h.shape[0]
    all_h = [hcur]
    for li in range(N_LAYER):
        lay = params["layers"][li]
        Ah = _lin(lay["A"], hcur)
        Bh = _lin(lay["B"], hcur)
        Dh = _lin(lay["D"], hcur)
        Eh = _lin(lay["E"], hcur)
        DSBS, ES = _gather2(jnp.concatenate([Dh, Bh], axis=1), Eh, src, dst)
        e_new, sigma, msg, st = _edge_pass1(
            DSBS, ES, ecur, lay["C"]["W"], lay["C"]["b"][None, :], snorm_e)
        num, den = _segsum2(msg, sigma, dst)
        hn = Ah + num / (den + 1e-6)
        hn = _bn(hn * snorm_n, lay["bn_h_g"], lay["bn_h_b"])
        hn = hcur + jax.nn.relu(hn)
        m = st[0] / E
        v = st[1] / E - m * m
        A = lay["bn_e_g"] / jnp.sqrt(v + 1e-5)
        B = lay["bn_e_b"] - m * A
        en = ecur + jax.nn.relu(e_new * snorm_e * A[None, :] + B[None, :])
        all_h.append(hn)
        hcur = jax.nn.relu(_lin(params["dense"][li], jnp.concatenate(all_h, axis=1)))
        ecur = en
    return _mlp_head(hcur, params["mlp"])


# speedup vs baseline: 2.8379x; 1.0101x over previous
import functools

import jax
import jax.numpy as jnp
from jax import lax
from jax.experimental import pallas as pl
from jax.experimental.pallas import tpu as pltpu
from jax.experimental.pallas import tpu_sc as plsc

N = 10000
E = 320000
H = 128
L = 20
N_CLASS = 10
N_LAYER = 3


_TIL = 16
_EPT = E // _TIL
_CH = 512
_NCH = _EPT // _CH
_TAIL = _EPT - _NCH * _CH
_NHALF = 5120
_DUMP = _NHALF
_ROWS_T = _NHALF // _TIL


def _remap(buf, n, base_n):
    for v in range(n // 16):
        d = buf[pl.ds(v * 16, 16)] - base_n
        ok = (d >= 0) & (d < _NHALF)
        buf[pl.ds(v * 16, 16)] = jnp.where(ok, d, _DUMP)


def _segsum2_kernel(msg_hbm, sig_hbm, idx_hbm, z_hbm, num_hbm, den_hbm,
                    vals_v, i0, i1, i2, i3, it, accum, sem):
    c = lax.axis_index("c")
    s = lax.axis_index("s")
    base_n = c * _NHALF
    idx_bufs = (i0, i1, i2, i3)

    def zero_accum():
        pltpu.sync_copy(z_hbm, accum.at[pl.ds(s * _ROWS_T, _ROWS_T)])

    def run(vhbm):
        base = s * _EPT

        def chunk(k, carry):
            cb = base + k * _CH
            ds_ = [pltpu.async_copy(vhbm.at[pl.ds(cb, _CH)], vals_v, sem)]
            for j in range(4):
                ds_.append(pltpu.async_copy(
                    idx_hbm.at[pl.ds(cb + j * 128, 128)], idx_bufs[j], sem))
            for d in ds_:
                d.wait()
            for j in range(4):
                _remap(idx_bufs[j], 128, base_n)
            sc = [pltpu.async_copy(vals_v.at[pl.ds(j * 128, 128)],
                                   accum.at[idx_bufs[j]], sem, add=True)
                  for j in range(4)]
            for d in sc:
                d.wait()
            return carry

        lax.fori_loop(0, _NCH, chunk, 0)
        tb = base + _NCH * _CH
        pltpu.sync_copy(vhbm.at[pl.ds(tb, _TAIL)], vals_v.at[pl.ds(0, _TAIL)])
        pltpu.sync_copy(idx_hbm.at[pl.ds(tb, _TAIL)], it)
        _remap(it, _TAIL, base_n)
        pltpu.sync_copy(vals_v.at[pl.ds(0, _TAIL)], accum.at[it], add=True)

    def copy_out(dst_hbm):
        rb = s * _ROWS_T
        pltpu.sync_copy(accum.at[pl.ds(rb, _ROWS_T)],
                        vals_v.at[pl.ds(0, _ROWS_T)])
        pltpu.sync_copy(vals_v.at[pl.ds(0, _ROWS_T)],
                        dst_hbm.at[pl.ds(base_n + rb, _ROWS_T)])

    zero_accum()
    plsc.subcore_barrier()
    run(msg_hbm)
    plsc.subcore_barrier()
    copy_out(num_hbm)
    plsc.subcore_barrier()
    zero_accum()
    plsc.subcore_barrier()
    run(sig_hbm)
    plsc.subcore_barrier()
    copy_out(den_hbm)


@jax.jit
def _segsum2(msg, sig, dst):
    mesh = plsc.VectorSubcoreMesh(core_axis_name="c", subcore_axis_name="s")
    zeros = jnp.zeros((_ROWS_T, H), jnp.float32)
    k = functools.partial(
        pl.kernel,
        mesh=mesh,
        out_type=(jax.ShapeDtypeStruct((2 * _NHALF, H), jnp.float32),
                  jax.ShapeDtypeStruct((2 * _NHALF, H), jnp.float32)),
        scratch_types=[
            pltpu.VMEM((_CH, H), jnp.float32),
            pltpu.VMEM((128,), jnp.int32),
            pltpu.VMEM((128,), jnp.int32),
            pltpu.VMEM((128,), jnp.int32),
            pltpu.VMEM((128,), jnp.int32),
            pltpu.VMEM((_TAIL,), jnp.int32),
            pltpu.VMEM_SHARED((_NHALF + 8, H), jnp.float32),
            pltpu.SemaphoreType.DMA,
        ],
    )(_segsum2_kernel)
    num, den = k(msg, sig, dst, zeros)
    return num[:N], den[:N]


_GW = 32
_GEPW = E // _GW
_GCH = 128
_GNCH = _GEPW // _GCH
_GTAIL = _GEPW - _GNCH * _GCH


def _gather2_kernel(db_hbm, eh_hbm, src_hbm, dst_hbm,
                    dsbs_hbm, es_hbm,
                    db0, eb0, s0, d0, db1, eb1, s1, d1, st, dt,
                    si0, sg0, sw0, si1, sg1, sw1):
    c = lax.axis_index("c")
    s = lax.axis_index("s")
    w = s * 2 + c
    base = w * _GEPW
    sets = ((db0, eb0, s0, d0, si0, sg0, sw0),
            (db1, eb1, s1, d1, si1, sg1, sw1))

    def do_chunk(k, p, first):
        dbuf, ebuf, sb, db_, si, sg, sw = sets[p]
        cb = base + k * _GCH
        lds = [pltpu.async_copy(src_hbm.at[pl.ds(cb, _GCH)], sb, si),
               pltpu.async_copy(dst_hbm.at[pl.ds(cb, _GCH)], db_, si)]
        if not first:
            pltpu.make_async_copy(db_hbm.at[pl.ds(0, _GCH)], dbuf, sw).wait()
            pltpu.make_async_copy(eh_hbm.at[pl.ds(0, _GCH)], ebuf, sw).wait()
        for d in lds:
            d.wait()
        gs = [pltpu.async_copy(db_hbm.at[sb], dbuf, sg),
              pltpu.async_copy(eh_hbm.at[db_], ebuf, sg)]
        for d in gs:
            d.wait()
        pltpu.async_copy(dbuf, dsbs_hbm.at[pl.ds(cb, _GCH)], sw)
        pltpu.async_copy(ebuf, es_hbm.at[pl.ds(cb, _GCH)], sw)

    do_chunk(0, 0, True)
    do_chunk(1, 1, True)

    def pair(m, carry):
        do_chunk(2 * m, 0, False)
        do_chunk(2 * m + 1, 1, False)
        return carry

    lax.fori_loop(1, _GNCH // 2, pair, 0)
    for p in range(2):
        dbuf, ebuf, sw = sets[p][0], sets[p][1], sets[p][6]
        pltpu.make_async_copy(db_hbm.at[pl.ds(0, _GCH)], dbuf, sw).wait()
        pltpu.make_async_copy(eh_hbm.at[pl.ds(0, _GCH)], ebuf, sw).wait()
    tb = base + _GNCH * _GCH
    pltpu.sync_copy(src_hbm.at[pl.ds(tb, _GTAIL)], st)
    pltpu.sync_copy(dst_hbm.at[pl.ds(tb, _GTAIL)], dt)
    pltpu.sync_copy(db_hbm.at[st], db0.at[pl.ds(0, _GTAIL)])
    pltpu.sync_copy(eh_hbm.at[dt], eb0.at[pl.ds(0, _GTAIL)])
    pltpu.sync_copy(db0.at[pl.ds(0, _GTAIL)], dsbs_hbm.at[pl.ds(tb, _GTAIL)])
    pltpu.sync_copy(eb0.at[pl.ds(0, _GTAIL)], es_hbm.at[pl.ds(tb, _GTAIL)])


@jax.jit
def _gather2(dh_bh, eh, src, dst):
    mesh = plsc.VectorSubcoreMesh(core_axis_name="c", subcore_axis_name="s")
    k = functools.partial(
        pl.kernel,
        mesh=mesh,
        out_type=(jax.ShapeDtypeStruct((E, 2 * H), jnp.float32),
                  jax.ShapeDtypeStruct((E, H), jnp.float32)),
        scratch_types=[
            pltpu.VMEM((_GCH, 2 * H), jnp.float32),
            pltpu.VMEM((_GCH, H), jnp.float32),
            pltpu.VMEM((128,), jnp.int32),
            pltpu.VMEM((128,), jnp.int32),
            pltpu.VMEM((_GCH, 2 * H), jnp.float32),
            pltpu.VMEM((_GCH, H), jnp.float32),
            pltpu.VMEM((128,), jnp.int32),
            pltpu.VMEM((128,), jnp.int32),
            pltpu.VMEM((_GTAIL,), jnp.int32),
            pltpu.VMEM((_GTAIL,), jnp.int32),
            pltpu.SemaphoreType.DMA,
            pltpu.SemaphoreType.DMA,
            pltpu.SemaphoreType.DMA,
            pltpu.SemaphoreType.DMA,
            pltpu.SemaphoreType.DMA,
            pltpu.SemaphoreType.DMA,
        ],
    )(_gather2_kernel)
    return k(dh_bh, eh, src, dst)


_TGW = 32
_TIDS = 204800
_TPW = _TIDS // _TGW
_TCH = 256
_TNCH = _TPW // _TCH


def _tgather_kernel(tab_hbm, ids_hbm, out_hbm, rbuf, i0, i1, sem):
    c = lax.axis_index("c")
    s = lax.axis_index("s")
    base = (s * 2 + c) * _TPW
    ibufs = (i0, i1)

    def chunk(k, carry):
        cb = base + k * _TCH
        ds_ = [pltpu.async_copy(ids_hbm.at[pl.ds(cb + j * 128, 128)],
                                ibufs[j], sem) for j in range(2)]
        for d in ds_:
            d.wait()
        gs = [pltpu.async_copy(tab_hbm.at[ibufs[j]],
                               rbuf.at[pl.ds(j * 128, 128)], sem)
              for j in range(2)]
        for d in gs:
            d.wait()
        pltpu.async_copy(rbuf, out_hbm.at[pl.ds(cb, _TCH)], sem).wait()
        return carry

    lax.fori_loop(0, _TNCH, chunk, 0)


@jax.jit
def _tgather(tab, ids):
    mesh = plsc.VectorSubcoreMesh(core_axis_name="c", subcore_axis_name="s")
    k = functools.partial(
        pl.kernel,
        mesh=mesh,
        out_type=jax.ShapeDtypeStruct((_TIDS, H), jnp.float32),
        scratch_types=[
            pltpu.VMEM((_TCH, H), jnp.float32),
            pltpu.VMEM((128,), jnp.int32),
            pltpu.VMEM((128,), jnp.int32),
            pltpu.SemaphoreType.DMA,
        ],
    )(_tgather_kernel)
    return k(tab, ids)


_EB = 2000
_ENB = E // _EB


def _edge1_kernel(dsbs_ref, es_ref, ec_ref, c_ref, bc_ref, sn_ref,
                  enew_ref, sig_ref, msg_ref, st_ref, acc):
    i = pl.program_id(0)

    @pl.when(i == 0)
    def _():
        acc[...] = jnp.zeros_like(acc)

    d = dsbs_ref[...]
    x = d[:, :H] + es_ref[...] + bc_ref[...]
    x = x + jnp.dot(ec_ref[...], c_ref[...], preferred_element_type=jnp.float32)
    sg = jax.nn.sigmoid(x)
    enew_ref[...] = x
    sig_ref[...] = sg
    msg_ref[...] = sg * d[:, H:]
    y = x * sn_ref[...]
    acc[0:1, :] += jnp.sum(y, axis=0, keepdims=True)
    acc[1:2, :] += jnp.sum(y * y, axis=0, keepdims=True)

    @pl.when(i == _ENB - 1)
    def _():
        st_ref[...] = acc[...]


@jax.jit
def _edge_pass1(dsbs, es, ecur, cw, cb, sn):
    sh = jax.ShapeDtypeStruct((E, H), jnp.float32)
    out = pl.pallas_call(
        _edge1_kernel,
        grid=(_ENB,),
        in_specs=[
            pl.BlockSpec((_EB, 2 * H), lambda i: (i, 0)),
            pl.BlockSpec((_EB, H), lambda i: (i, 0)),
            pl.BlockSpec((_EB, H), lambda i: (i, 0)),
            pl.BlockSpec((H, H), lambda i: (0, 0)),
            pl.BlockSpec((1, H), lambda i: (0, 0)),
            pl.BlockSpec((_EB, 1), lambda i: (i, 0)),
        ],
        out_specs=[
            pl.BlockSpec((_EB, H), lambda i: (i, 0)),
            pl.BlockSpec((_EB, H), lambda i: (i, 0)),
            pl.BlockSpec((_EB, H), lambda i: (i, 0)),
            pl.BlockSpec((8, H), lambda i: (0, 0)),
        ],
        out_shape=[sh, sh, sh, jax.ShapeDtypeStruct((8, H), jnp.float32)],
        scratch_shapes=[pltpu.VMEM((8, H), jnp.float32)],
    )(dsbs, es, ecur, cw, cb, sn)
    return out


_LB = 512
_NPAD = 10240


def _bilstm_kernel(x_ref, h_ref, len_ref,
                   wif_ref, whf_ref, bf_ref, wib_ref, whb_ref, bb_ref,
                   wh_ref, bh_ref, out_ref):
    x = x_ref[...]
    ln = len_ref[...]
    wif, whf, bf = wif_ref[...], whf_ref[...], bf_ref[...]
    wib, whb, bb = wib_ref[...], whb_ref[...], bb_ref[...]

    def step(xt, h, c, wi, wh, b):
        g = jnp.dot(xt, wi, preferred_element_type=jnp.float32)
        g = g + jnp.dot(h, wh, preferred_element_type=jnp.float32) + b
        i_ = jax.nn.sigmoid(g[:, 0:128])
        f_ = jax.nn.sigmoid(g[:, 128:256])
        g_ = jnp.tanh(g[:, 256:384])
        o_ = jax.nn.sigmoid(g[:, 384:512])
        c2 = f_ * c + i_ * g_
        h2 = o_ * jnp.tanh(c2)
        return h2, c2

    z = jnp.zeros((_LB, H), jnp.float32)
    h, c, hf = z, z, z
    for t in range(L):
        xt = x[:, t * H:(t + 1) * H]
        h, c = step(xt, h, c, wif, whf, bf)
        sel = (ln == t + 1).astype(jnp.float32)
        hf = hf + sel * h
    h, c = z, z
    for u in range(L - 1, -1, -1):
        xt = x[:, u * H:(u + 1) * H]
        h2, c2 = step(xt, h, c, wib, whb, bb)
        act = ln > u
        h = jnp.where(act, h2, h)
        c = jnp.where(act, c2, c)
    te = 0.5 * (hf + h)
    nrm = jnp.sqrt(jnp.sum(te * te, axis=1, keepdims=True))
    te = te / jnp.maximum(nrm, 1e-12)
    out_ref[...] = (
        jnp.dot(h_ref[...], wh_ref[...], preferred_element_type=jnp.float32)
        + bh_ref[...] + te)


@jax.jit
def _bilstm_hcur(txt2d, hpad, len2d, pf, pb, ph):
    grid = (_NPAD // _LB,)
    k = pl.pallas_call(
        _bilstm_kernel,
        grid=grid,
        in_specs=[
            pl.BlockSpec((_LB, L * H), lambda i: (i, 0)),
            pl.BlockSpec((_LB, H), lambda i: (i, 0)),
            pl.BlockSpec((_LB, 1), lambda i: (i, 0)),
            pl.BlockSpec((H, 4 * H), lambda i: (0, 0)),
            pl.BlockSpec((H, 4 * H), lambda i: (0, 0)),
            pl.BlockSpec((1, 4 * H), lambda i: (0, 0)),
            pl.BlockSpec((H, 4 * H), lambda i: (0, 0)),
            pl.BlockSpec((H, 4 * H), lambda i: (0, 0)),
            pl.BlockSpec((1, 4 * H), lambda i: (0, 0)),
            pl.BlockSpec((H, H), lambda i: (0, 0)),
            pl.BlockSpec((1, H), lambda i: (0, 0)),
        ],
        out_specs=pl.BlockSpec((_LB, H), lambda i: (i, 0)),
        out_shape=jax.ShapeDtypeStruct((_NPAD, H), jnp.float32),
    )
    return k(txt2d, hpad, len2d,
             pf["W_ih"].T, pf["W_hh"].T, pf["b"][None, :],
             pb["W_ih"].T, pb["W_hh"].T, pb["b"][None, :],
             ph["W"], ph["b"][None, :])


def _lin(p, x):
    return x @ p["W"] + p["b"]


def _bn(x, g, b):
    m = jnp.mean(x, axis=0)
    v = jnp.var(x, axis=0)
    return (x - m) / jnp.sqrt(v + 1e-5) * g + b


def _lstm(x, p):
    n = x.shape[0]
    xT = jnp.swapaxes(x, 0, 1)
    h0 = jnp.zeros((n, H), x.dtype)
    c0 = jnp.zeros((n, H), x.dtype)

    def step(carry, xt):
        hh, cc = carry
        gates = xt @ p["W_ih"].T + hh @ p["W_hh"].T + p["b"]
        i_, f_, g_, o_ = jnp.split(gates, 4, axis=-1)
        i_ = jax.nn.sigmoid(i_)
        f_ = jax.nn.sigmoid(f_)
        g_ = jnp.tanh(g_)
        o_ = jax.nn.sigmoid(o_)
        cc = f_ * cc + i_ * g_
        hh = o_ * jnp.tanh(cc)
        return (hh, cc), hh

    _, hs = jax.lax.scan(step, (h0, c0), xT)
    return jnp.swapaxes(hs, 0, 1)


def _mlp_head_kernel(y_ref, w0_ref, b0_ref, w1_ref, b1_ref, w2_ref, b2_ref,
                     out_ref):
    y = y_ref[...]
    y = jnp.maximum(y @ w0_ref[...] + b0_ref[...], 0.0)
    y = jnp.maximum(y @ w1_ref[...] + b1_ref[...], 0.0)
    out_ref[...] = y @ w2_ref[...] + b2_ref[...]


def _mlp_head(y, mlp):
    n = y.shape[0]
    blk = 2000
    grid = (n // blk,)
    w0, b0 = mlp[0]["W"], mlp[0]["b"]
    w1, b1 = mlp[1]["W"], mlp[1]["b"]
    w2, b2 = mlp[2]["W"], mlp[2]["b"]
    w2p = jnp.zeros((w2.shape[0], 128), w2.dtype).at[:, :N_CLASS].set(w2)
    b2p = jnp.zeros((128,), b2.dtype).at[:N_CLASS].set(b2)
    out = pl.pallas_call(
        _mlp_head_kernel,
        grid=grid,
        in_specs=[
            pl.BlockSpec((blk, H), lambda i: (i, 0)),
            pl.BlockSpec((H, 64), lambda i: (0, 0)),
            pl.BlockSpec((64,), lambda i: (0,)),
            pl.BlockSpec((64, 32), lambda i: (0, 0)),
            pl.BlockSpec((32,), lambda i: (0,)),
            pl.BlockSpec((32, 128), lambda i: (0, 0)),
            pl.BlockSpec((128,), lambda i: (0,)),
        ],
        out_specs=pl.BlockSpec((blk, 128), lambda i: (i, 0)),
        out_shape=jax.ShapeDtypeStruct((n, 128), y.dtype),
    )(y, w0, b0, w1, b1, w2p, b2p)
    return out[:, :N_CLASS]


def kernel(h, e, text, snorm_n, snorm_e, edge_index, text_length,
           graph_node_size, graph_edge_size, params):
    ids = jnp.zeros((_TIDS,), jnp.int32).at[:N * L].set(
        text.reshape(-1).astype(jnp.int32))
    txt2d = _tgather(params["text_emb"], ids).reshape(_NPAD, L * H)
    hpad = jnp.zeros((_NPAD, H), jnp.float32).at[:N].set(h)
    len2d = jnp.ones((_NPAD, 1), jnp.int32).at[:N, 0].set(text_length.astype(jnp.int32))
    hcur = _bilstm_hcur(txt2d, hpad, len2d,
                        params["lstm_f"], params["lstm_b"],
                        params["h_emb"])[:N]
    ecur = _lin(params["e_emb"], e)
    src = edge_index[0]
    dst = edge_index[1]
    n_nodes = h.shape[0]
    all_h = [hcur]
    for li in range(N_LAYER):
        lay = params["layers"][li]
        Ah = _lin(lay["A"], hcur)
        Bh = _lin(lay["B"], hcur)
        Dh = _lin(lay["D"], hcur)
        Eh = _lin(lay["E"], hcur)
        DSBS, ES = _gather2(jnp.concatenate([Dh, Bh], axis=1), Eh, src, dst)
        e_new, sigma, msg, st = _edge_pass1(
            DSBS, ES, ecur, lay["C"]["W"], lay["C"]["b"][None, :], snorm_e)
        num, den = _segsum2(msg, sigma, dst)
        hn = Ah + num / (den + 1e-6)
        hn = _bn(hn * snorm_n, lay["bn_h_g"], lay["bn_h_b"])
        hn = hcur + jax.nn.relu(hn)
        if li < N_LAYER - 1:
            m = st[0] / E
            v = st[1] / E - m * m
            A = lay["bn_e_g"] / jnp.sqrt(v + 1e-5)
            B = lay["bn_e_b"] - m * A
            en = ecur + jax.nn.relu(e_new * snorm_e * A[None, :] + B[None, :])
        else:
            en = ecur
        all_h.append(hn)
        hcur = jax.nn.relu(_lin(params["dense"][li], jnp.concatenate(all_h, axis=1)))
        ecur = en
    return _mlp_head(hcur, params["mlp"])
